# Initial kernel scaffold; baseline (speedup 1.0000x reference)
import jax
import jax.numpy as jnp
from jax.experimental import pallas as pl


def kernel(x, edge_index, edge_attr, batch_seg, params):
    raise NotImplementedError("write your pallas kernel here")


# speedup vs baseline: 1.7610x; 1.7610x over previous
import functools

import jax
import jax.numpy as jnp
from jax import lax
from jax.experimental import pallas as pl
from jax.experimental.pallas import tpu as pltpu
from jax.experimental.pallas import tpu_sc as plsc

_N = 5000
_E = 10000
_NG = 256
_NT = 100

_NC = 2
_NS = 16
_NW = _NC * _NS
_CH = 80
_NCHUNK = 4
_TILE_E = _CH * _NCHUNK
_EP = _NW * _TILE_E
_NP = _NS * _TILE_E
_EB = 512
_NB = 256


def _sc_mesh():
    return plsc.VectorSubcoreMesh(core_axis_name="c", subcore_axis_name="s")


def _gather_call(h, src3):
    cinp = 128

    @functools.partial(
        pl.kernel,
        out_type=jax.ShapeDtypeStruct((_EP, cinp), jnp.float32),
        mesh=_sc_mesh(),
        scratch_types=[
            pltpu.VMEM((_NCHUNK, _CH), jnp.int32),
            pltpu.VMEM((_CH, cinp), jnp.float32),
            pltpu.SemaphoreType.DMA,
        ],
    )
    def k(h_hbm, src_hbm, out_hbm, idx_v, row_v, sem):
        c = lax.axis_index("c")
        s = lax.axis_index("s")
        wid = s * _NC + c
        pltpu.sync_copy(src_hbm.at[wid], idx_v)
        for j in range(_NCHUNK):
            pltpu.async_copy(h_hbm.at[idx_v.at[j]], row_v, sem).wait()
            pltpu.sync_copy(row_v, out_hbm.at[pl.ds(wid * _TILE_E + j * _CH, _CH)])

    return k(h, src3)


_EC = _EP // _NC
_CS = 16
_MCH = 1024


def _scatter_call(msgt, dst2, zrows):
    @functools.partial(
        pl.kernel,
        out_type=jax.ShapeDtypeStruct((2 * 256, _NP), jnp.float32),
        mesh=_sc_mesh(),
        compiler_params=pltpu.CompilerParams(needs_layout_passes=False),
        scratch_types=[
            pltpu.VMEM((_EC,), jnp.int32),
            pltpu.VMEM((_CS, _MCH), jnp.float32),
            pltpu.VMEM((_CS, _NP), jnp.float32),
        ],
    )
    def k(msg_hbm, dst_hbm, zero_hbm, out_hbm, dstv, mbuf, table):
        c = lax.axis_index("c")
        s = lax.axis_index("s")
        rows = lax.iota(jnp.int32, 16)
        for t in range(_NP // 512):
            pltpu.sync_copy(zero_hbm, table.at[:, pl.ds(t * 512, 512)])
        pltpu.sync_copy(dst_hbm.at[c], dstv)
        for t in range(_EC // _MCH):
            pltpu.sync_copy(
                msg_hbm.at[pl.ds(s * _CS, _CS), pl.ds(c * _EC + t * _MCH, _MCH)],
                mbuf,
            )

            def grp(i, _):
                for l in range(16):
                    pos = t * _MCH + i * 16 + l
                    d = plsc.load_gather(
                        dstv, [jnp.broadcast_to(pos, (16,)).astype(jnp.int32)]
                    )
                    e = jnp.broadcast_to(i * 16 + l, (16,)).astype(jnp.int32)
                    val = plsc.load_gather(mbuf, [rows, e])
                    plsc.addupdate_scatter(table, [rows, d], val)
                return 0

            lax.fori_loop(0, _MCH // 16, grp, jnp.int32(0))
        pltpu.sync_copy(table, out_hbm.at[pl.ds(c * 256 + s * _CS, _CS)])

    return k(msgt, dst2, zrows)


def _msg_call(eap, hsrc, w1p, b1r, w2r, b2r, cinp, cout, ones_cols):
    nk = 32
    wtot = 256

    def body(ea_ref, hs_ref, w1_ref, b1_ref, w2_ref, b2_ref, out_ref):
        eh = jnp.maximum(
            jnp.dot(ea_ref[...], w1_ref[...], preferred_element_type=jnp.float32)
            + b1_ref[...],
            0.0,
        )
        hs = hs_ref[...][:, :cinp]
        q = jnp.concatenate([eh[:, k : k + 1] * hs for k in range(nk)], axis=1)
        msg = jnp.dot(q, w2_ref[...], preferred_element_type=jnp.float32) + jnp.dot(
            hs, b2_ref[...], preferred_element_type=jnp.float32
        )
        pieces = [msg]
        if ones_cols:
            pieces.append(jnp.ones((msg.shape[0], ones_cols), jnp.float32))
        pad = wtot - cout - ones_cols
        if pad:
            pieces.append(jnp.zeros((msg.shape[0], pad), jnp.float32))
        full = jnp.concatenate(pieces, axis=1) if len(pieces) > 1 else msg
        out_ref[...] = full.T

    return pl.pallas_call(
        body,
        grid=(_EP // _EB,),
        in_specs=[
            pl.BlockSpec((_EB, 8), lambda i: (i, 0)),
            pl.BlockSpec((_EB, 128), lambda i: (i, 0)),
            pl.BlockSpec((8, 32), lambda i: (0, 0)),
            pl.BlockSpec((1, 32), lambda i: (0, 0)),
            pl.BlockSpec((nk * cinp, cout), lambda i: (0, 0)),
            pl.BlockSpec((cinp, cout), lambda i: (0, 0)),
        ],
        out_specs=pl.BlockSpec((wtot, _EB), lambda i: (0, i)),
        out_shape=jax.ShapeDtypeStruct((wtot, _EP), jnp.float32),
    )(eap, hsrc, w1p, b1r, w2r, b2r)


def _node_call(h, rootp, parts, inv_or_cnt, biasr, gammar, betar, cinp, cout, first):
    wout = max(cout, 128)
    nblk = _NP // _NB

    def body(h_ref, root_ref, p0_ref, p1_ref, cv_ref, bias_ref,
             g_ref, beta_ref, out_ref, inv_ref):
        p0t = p0_ref[...]
        p1t = p1_ref[...]
        psum = (p0t[:cout, :] + p1t[:cout, :]).T
        if first:
            cntt = p0t[cout : cout + 16, :] + p1t[cout : cout + 16, :]
            cnt = cntt.T
            inv = 1.0 / jnp.maximum(cnt[:, :1], 1.0)
            inv_ref[...] = jnp.broadcast_to(inv, (_NB, 16))
        else:
            inv = cv_ref[...][:, :1]
        agg = psum * inv
        y = (
            jnp.dot(h_ref[...], root_ref[...], preferred_element_type=jnp.float32)
            + agg
            + bias_ref[...]
        )
        hv = jnp.maximum(y * g_ref[...] + beta_ref[...], 0.0)
        if wout > cout:
            hv = jnp.concatenate(
                [hv, jnp.zeros((_NB, wout - cout), jnp.float32)], axis=1
            )
        out_ref[...] = hv

    in_specs = [
        pl.BlockSpec((_NB, 128), lambda i: (i, 0)),
        pl.BlockSpec((128, cout), lambda i: (0, 0)),
        pl.BlockSpec((256, _NB), lambda i: (0, i)),
        pl.BlockSpec((256, _NB), lambda i: (1, i)),
        pl.BlockSpec((_NB, 16), lambda i: (i, 0)),
        pl.BlockSpec((1, cout), lambda i: (0, 0)),
        pl.BlockSpec((1, cout), lambda i: (0, 0)),
        pl.BlockSpec((1, cout), lambda i: (0, 0)),
    ]
    inv_in = jnp.zeros((_NP, 16), jnp.float32) if first else inv_or_cnt
    out = pl.pallas_call(
        body,
        grid=(nblk,),
        in_specs=in_specs,
        out_specs=[
            pl.BlockSpec((_NB, wout), lambda i: (i, 0)),
            pl.BlockSpec((_NB, 16), lambda i: (i, 0)),
        ],
        out_shape=[
            jax.ShapeDtypeStruct((_NP, wout), jnp.float32),
            jax.ShapeDtypeStruct((_NP, 16), jnp.float32),
        ],
    )(h, rootp, parts, parts, inv_in, biasr, gammar, betar)
    return out


def _pool_call(h3, bs3, wpp, bpp):
    nblk = _NP // _NB

    def body(h_ref, bs_ref, wp_ref, bp_ref, out_ref, acc, pcnt):
        i = pl.program_id(0)

        @pl.when(i == 0)
        def _init():
            acc[...] = jnp.zeros_like(acc)
            pcnt[...] = jnp.zeros_like(pcnt)

        seg = lax.broadcasted_iota(jnp.int32, (_NG, _NB), 0)
        bs = bs_ref[0]
        oh = (seg == bs).astype(jnp.float32)
        acc[...] += jnp.dot(oh, h_ref[...], preferred_element_type=jnp.float32)
        pcnt[...] += jnp.broadcast_to(
            jnp.sum(oh, axis=1, keepdims=True), (_NG, 128)
        )

        @pl.when(i == nblk - 1)
        def _fin():
            pooled = acc[...] * (1.0 / jnp.maximum(pcnt[...][:, :1], 1.0))
            o = jnp.dot(
                pooled, wp_ref[...], preferred_element_type=jnp.float32
            ) + bp_ref[...]
            out_ref[...] = jnp.where(o > 0, o, 0.1 * o)

    return pl.pallas_call(
        body,
        grid=(nblk,),
        in_specs=[
            pl.BlockSpec((_NB, 256), lambda i: (i, 0)),
            pl.BlockSpec((1, 1, _NB), lambda i: (i, 0, 0)),
            pl.BlockSpec((256, 128), lambda i: (0, 0)),
            pl.BlockSpec((1, 128), lambda i: (0, 0)),
        ],
        out_specs=pl.BlockSpec((_NG, 128), lambda i: (0, 0)),
        out_shape=jax.ShapeDtypeStruct((_NG, 128), jnp.float32),
        scratch_shapes=[
            pltpu.VMEM((_NG, 256), jnp.float32),
            pltpu.VMEM((_NG, 128), jnp.float32),
        ],
    )(h3, bs3, wpp, bpp)


def _prep_layer(p, cin, cinp, cout):
    w1p = jnp.zeros((8, 32), jnp.float32).at[:3].set(p["W1"])
    b1r = p["b1"].reshape(1, 32)
    w2 = p["W2"].reshape(32, cin, cout)
    w2r = (
        jnp.zeros((32, cinp, cout), jnp.float32)
        .at[:, :cin, :]
        .set(w2)
        .reshape(32 * cinp, cout)
    )
    b2r = jnp.zeros((cinp, cout), jnp.float32).at[:cin].set(
        p["b2"].reshape(cin, cout)
    )
    rootp = jnp.zeros((128, cout), jnp.float32).at[:cin].set(p["root"])
    biasr = p["bias"].reshape(1, cout)
    gammar = (p["gamma"] / jnp.sqrt(1.0 + 1e-5)).reshape(1, cout)
    betar = p["beta"].reshape(1, cout)
    return w1p, b1r, w2r, b2r, rootp, biasr, gammar, betar


def kernel(x, edge_index, edge_attr, batch_seg, params):
    f32 = jnp.float32
    src = edge_index[0]
    dst = edge_index[1]
    xp = jnp.zeros((_NP, 128), f32).at[:_N, :5].set(x)
    src3 = (
        jnp.zeros((_EP,), jnp.int32).at[:_E].set(src).reshape(_NW, _NCHUNK, _CH)
    )
    dst2 = jnp.full((_EP,), _N, jnp.int32).at[:_E].set(dst).reshape(_NC, _EC)
    eap = jnp.zeros((_EP, 8), f32).at[:_E, :3].set(edge_attr)
    bs3 = (
        jnp.full((_NP,), _NG + 8, jnp.int32)
        .at[:_N]
        .set(batch_seg)
        .reshape(_NP // _NB, 1, _NB)
    )
    zrows = jnp.zeros((_CS, 512), f32)
    l1 = _prep_layer(params["layer1"], 5, 16, 64)
    l2 = _prep_layer(params["layer2"], 64, 64, 128)
    l3 = _prep_layer(params["layer3"], 128, 128, 256)
    wpp = jnp.zeros((256, 128), f32).at[:, :_NT].set(params["mlp_W"])
    bpp = jnp.zeros((1, 128), f32).at[0, :_NT].set(params["mlp_b"])

    w1p, b1r, w2r, b2r, rootp, biasr, gammar, betar = l1
    hs = _gather_call(xp, src3)
    msg = _msg_call(eap, hs, w1p, b1r, w2r, b2r, 16, 64, 16)
    parts = _scatter_call(msg, dst2, zrows)
    h, inv = _node_call(xp, rootp, parts, None, biasr, gammar, betar, 16, 64, True)

    w1p, b1r, w2r, b2r, rootp, biasr, gammar, betar = l2
    hs = _gather_call(h, src3)
    msg = _msg_call(eap, hs, w1p, b1r, w2r, b2r, 64, 128, 0)
    parts = _scatter_call(msg, dst2, zrows)
    h, _ = _node_call(h, rootp, parts, inv, biasr, gammar, betar, 64, 128, False)

    w1p, b1r, w2r, b2r, rootp, biasr, gammar, betar = l3
    hs = _gather_call(h, src3)
    msg = _msg_call(eap, hs, w1p, b1r, w2r, b2r, 128, 256, 0)
    parts = _scatter_call(msg, dst2, zrows)
    h, _ = _node_call(h, rootp, parts, inv, biasr, gammar, betar, 128, 256, False)

    out = _pool_call(h, bs3, wpp, bpp)
    return out[:, :_NT]


# speedup vs baseline: 2.6963x; 1.5311x over previous
import functools

import jax
import jax.numpy as jnp
from jax import lax
from jax.experimental import pallas as pl
from jax.experimental.pallas import tpu as pltpu
from jax.experimental.pallas import tpu_sc as plsc

_N = 5000
_E = 10000
_NG = 256
_NT = 100

_NC = 2
_NS = 16
_NW = _NC * _NS
_CH = 80
_NCHUNK = 4
_TILE_E = _CH * _NCHUNK
_EP = _NW * _TILE_E
_NP = _NS * _TILE_E
_EB = 512
_NB = 256


def _sc_mesh():
    return plsc.VectorSubcoreMesh(core_axis_name="c", subcore_axis_name="s")


def _gather_call(h, src3):
    cinp = 128

    @functools.partial(
        pl.kernel,
        out_type=jax.ShapeDtypeStruct((_EP, cinp), jnp.float32),
        mesh=_sc_mesh(),
        scratch_types=[
            pltpu.VMEM((_NCHUNK, _CH), jnp.int32),
            pltpu.VMEM((_CH, cinp), jnp.float32),
            pltpu.SemaphoreType.DMA,
        ],
    )
    def k(h_hbm, src_hbm, out_hbm, idx_v, row_v, sem):
        c = lax.axis_index("c")
        s = lax.axis_index("s")
        wid = s * _NC + c
        pltpu.sync_copy(src_hbm.at[wid], idx_v)
        for j in range(_NCHUNK):
            pltpu.async_copy(h_hbm.at[idx_v.at[j]], row_v, sem).wait()
            pltpu.sync_copy(row_v, out_hbm.at[pl.ds(wid * _TILE_E + j * _CH, _CH)])

    return k(h, src3)


_EC = _EP // _NC
_CS = 16
_MCH = 1024


def _scatter_call(msgt, dst2, zrows):
    @functools.partial(
        pl.kernel,
        out_type=jax.ShapeDtypeStruct((2 * 256, _NP), jnp.float32),
        mesh=_sc_mesh(),
        compiler_params=pltpu.CompilerParams(needs_layout_passes=False),
        scratch_types=[
            pltpu.VMEM((_EC,), jnp.int32),
            pltpu.VMEM((_CS, _MCH), jnp.float32),
            pltpu.VMEM((_CS, _NP), jnp.float32),
        ],
    )
    def k(msg_hbm, dst_hbm, zero_hbm, out_hbm, dstv, mbuf, table):
        c = lax.axis_index("c")
        s = lax.axis_index("s")
        rows = lax.iota(jnp.int32, 16)
        for t in range(_NP // 512):
            pltpu.sync_copy(zero_hbm, table.at[:, pl.ds(t * 512, 512)])
        pltpu.sync_copy(dst_hbm.at[c], dstv)
        for t in range(_EC // _MCH):
            pltpu.sync_copy(
                msg_hbm.at[pl.ds(s * _CS, _CS), pl.ds(c * _EC + t * _MCH, _MCH)],
                mbuf,
            )

            def grp(i, _):
                d16 = dstv[pl.ds(t * _MCH + i * 16, 16)]
                for r in range(16):
                    vals = mbuf[r, pl.ds(i * 16, 16)]
                    rr = jnp.full((16,), r, jnp.int32)
                    plsc.addupdate_scatter(table, [rr, d16], vals)
                return _

            lax.fori_loop(0, _MCH // 16, grp, jnp.int32(0))
        pltpu.sync_copy(table, out_hbm.at[pl.ds(c * 256 + s * _CS, _CS)])

    return k(msgt, dst2, zrows)


def _msg_call(eap, hsrc, w1p, b1r, w2r, b2r, cinp, cout, ones_cols):
    nk = 32
    wtot = 256

    def body(ea_ref, hs_ref, w1_ref, b1_ref, w2_ref, b2_ref, out_ref):
        eh = jnp.maximum(
            jnp.dot(ea_ref[...], w1_ref[...], preferred_element_type=jnp.float32)
            + b1_ref[...],
            0.0,
        )
        hs = hs_ref[...][:, :cinp]
        q = jnp.concatenate([eh[:, k : k + 1] * hs for k in range(nk)], axis=1)
        msg = jnp.dot(q, w2_ref[...], preferred_element_type=jnp.float32) + jnp.dot(
            hs, b2_ref[...], preferred_element_type=jnp.float32
        )
        pieces = [msg]
        if ones_cols:
            pieces.append(jnp.ones((msg.shape[0], ones_cols), jnp.float32))
        pad = wtot - cout - ones_cols
        if pad:
            pieces.append(jnp.zeros((msg.shape[0], pad), jnp.float32))
        full = jnp.concatenate(pieces, axis=1) if len(pieces) > 1 else msg
        out_ref[...] = full.T

    return pl.pallas_call(
        body,
        grid=(_EP // _EB,),
        in_specs=[
            pl.BlockSpec((_EB, 8), lambda i: (i, 0)),
            pl.BlockSpec((_EB, 128), lambda i: (i, 0)),
            pl.BlockSpec((8, 32), lambda i: (0, 0)),
            pl.BlockSpec((1, 32), lambda i: (0, 0)),
            pl.BlockSpec((nk * cinp, cout), lambda i: (0, 0)),
            pl.BlockSpec((cinp, cout), lambda i: (0, 0)),
        ],
        out_specs=pl.BlockSpec((wtot, _EB), lambda i: (0, i)),
        out_shape=jax.ShapeDtypeStruct((wtot, _EP), jnp.float32),
    )(eap, hsrc, w1p, b1r, w2r, b2r)


def _node_call(h, rootp, parts, inv_or_cnt, biasr, gammar, betar, cinp, cout, first):
    wout = max(cout, 128)
    nblk = _NP // _NB

    def body(h_ref, root_ref, p0_ref, p1_ref, cv_ref, bias_ref,
             g_ref, beta_ref, out_ref, inv_ref):
        p0t = p0_ref[...]
        p1t = p1_ref[...]
        psum = (p0t[:cout, :] + p1t[:cout, :]).T
        if first:
            cntt = p0t[cout : cout + 16, :] + p1t[cout : cout + 16, :]
            cnt = cntt.T
            inv = 1.0 / jnp.maximum(cnt[:, :1], 1.0)
            inv_ref[...] = jnp.broadcast_to(inv, (_NB, 16))
        else:
            inv = cv_ref[...][:, :1]
        agg = psum * inv
        y = (
            jnp.dot(h_ref[...], root_ref[...], preferred_element_type=jnp.float32)
            + agg
            + bias_ref[...]
        )
        hv = jnp.maximum(y * g_ref[...] + beta_ref[...], 0.0)
        if wout > cout:
            hv = jnp.concatenate(
                [hv, jnp.zeros((_NB, wout - cout), jnp.float32)], axis=1
            )
        out_ref[...] = hv

    in_specs = [
        pl.BlockSpec((_NB, 128), lambda i: (i, 0)),
        pl.BlockSpec((128, cout), lambda i: (0, 0)),
        pl.BlockSpec((256, _NB), lambda i: (0, i)),
        pl.BlockSpec((256, _NB), lambda i: (1, i)),
        pl.BlockSpec((_NB, 16), lambda i: (i, 0)),
        pl.BlockSpec((1, cout), lambda i: (0, 0)),
        pl.BlockSpec((1, cout), lambda i: (0, 0)),
        pl.BlockSpec((1, cout), lambda i: (0, 0)),
    ]
    inv_in = jnp.zeros((_NP, 16), jnp.float32) if first else inv_or_cnt
    out = pl.pallas_call(
        body,
        grid=(nblk,),
        in_specs=in_specs,
        out_specs=[
            pl.BlockSpec((_NB, wout), lambda i: (i, 0)),
            pl.BlockSpec((_NB, 16), lambda i: (i, 0)),
        ],
        out_shape=[
            jax.ShapeDtypeStruct((_NP, wout), jnp.float32),
            jax.ShapeDtypeStruct((_NP, 16), jnp.float32),
        ],
    )(h, rootp, parts, parts, inv_in, biasr, gammar, betar)
    return out


def _pool_call(h3, bs3, wpp, bpp):
    nblk = _NP // _NB

    def body(h_ref, bs_ref, wp_ref, bp_ref, out_ref, acc, pcnt):
        i = pl.program_id(0)

        @pl.when(i == 0)
        def _init():
            acc[...] = jnp.zeros_like(acc)
            pcnt[...] = jnp.zeros_like(pcnt)

        seg = lax.broadcasted_iota(jnp.int32, (_NG, _NB), 0)
        bs = bs_ref[0]
        oh = (seg == bs).astype(jnp.float32)
        acc[...] += jnp.dot(oh, h_ref[...], preferred_element_type=jnp.float32)
        pcnt[...] += jnp.broadcast_to(
            jnp.sum(oh, axis=1, keepdims=True), (_NG, 128)
        )

        @pl.when(i == nblk - 1)
        def _fin():
            pooled = acc[...] * (1.0 / jnp.maximum(pcnt[...][:, :1], 1.0))
            o = jnp.dot(
                pooled, wp_ref[...], preferred_element_type=jnp.float32
            ) + bp_ref[...]
            out_ref[...] = jnp.where(o > 0, o, 0.1 * o)

    return pl.pallas_call(
        body,
        grid=(nblk,),
        in_specs=[
            pl.BlockSpec((_NB, 256), lambda i: (i, 0)),
            pl.BlockSpec((1, 1, _NB), lambda i: (i, 0, 0)),
            pl.BlockSpec((256, 128), lambda i: (0, 0)),
            pl.BlockSpec((1, 128), lambda i: (0, 0)),
        ],
        out_specs=pl.BlockSpec((_NG, 128), lambda i: (0, 0)),
        out_shape=jax.ShapeDtypeStruct((_NG, 128), jnp.float32),
        scratch_shapes=[
            pltpu.VMEM((_NG, 256), jnp.float32),
            pltpu.VMEM((_NG, 128), jnp.float32),
        ],
    )(h3, bs3, wpp, bpp)


def _prep_layer(p, cin, cinp, cout):
    w1p = jnp.zeros((8, 32), jnp.float32).at[:3].set(p["W1"])
    b1r = p["b1"].reshape(1, 32)
    w2 = p["W2"].reshape(32, cin, cout)
    w2r = (
        jnp.zeros((32, cinp, cout), jnp.float32)
        .at[:, :cin, :]
        .set(w2)
        .reshape(32 * cinp, cout)
    )
    b2r = jnp.zeros((cinp, cout), jnp.float32).at[:cin].set(
        p["b2"].reshape(cin, cout)
    )
    rootp = jnp.zeros((128, cout), jnp.float32).at[:cin].set(p["root"])
    biasr = p["bias"].reshape(1, cout)
    gammar = (p["gamma"] / jnp.sqrt(1.0 + 1e-5)).reshape(1, cout)
    betar = p["beta"].reshape(1, cout)
    return w1p, b1r, w2r, b2r, rootp, biasr, gammar, betar


def kernel(x, edge_index, edge_attr, batch_seg, params):
    f32 = jnp.float32
    src = edge_index[0]
    dst = edge_index[1]
    xp = jnp.zeros((_NP, 128), f32).at[:_N, :5].set(x)
    src3 = (
        jnp.zeros((_EP,), jnp.int32).at[:_E].set(src).reshape(_NW, _NCHUNK, _CH)
    )
    dst2 = jnp.full((_EP,), _N, jnp.int32).at[:_E].set(dst).reshape(_NC, _EC)
    eap = jnp.zeros((_EP, 8), f32).at[:_E, :3].set(edge_attr)
    bs3 = (
        jnp.full((_NP,), _NG + 8, jnp.int32)
        .at[:_N]
        .set(batch_seg)
        .reshape(_NP // _NB, 1, _NB)
    )
    zrows = jnp.zeros((_CS, 512), f32)
    l1 = _prep_layer(params["layer1"], 5, 16, 64)
    l2 = _prep_layer(params["layer2"], 64, 64, 128)
    l3 = _prep_layer(params["layer3"], 128, 128, 256)
    wpp = jnp.zeros((256, 128), f32).at[:, :_NT].set(params["mlp_W"])
    bpp = jnp.zeros((1, 128), f32).at[0, :_NT].set(params["mlp_b"])

    w1p, b1r, w2r, b2r, rootp, biasr, gammar, betar = l1
    hs = _gather_call(xp, src3)
    msg = _msg_call(eap, hs, w1p, b1r, w2r, b2r, 16, 64, 16)
    parts = _scatter_call(msg, dst2, zrows)
    h, inv = _node_call(xp, rootp, parts, None, biasr, gammar, betar, 16, 64, True)

    w1p, b1r, w2r, b2r, rootp, biasr, gammar, betar = l2
    hs = _gather_call(h, src3)
    msg = _msg_call(eap, hs, w1p, b1r, w2r, b2r, 64, 128, 0)
    parts = _scatter_call(msg, dst2, zrows)
    h, _ = _node_call(h, rootp, parts, inv, biasr, gammar, betar, 64, 128, False)

    w1p, b1r, w2r, b2r, rootp, biasr, gammar, betar = l3
    hs = _gather_call(h, src3)
    msg = _msg_call(eap, hs, w1p, b1r, w2r, b2r, 128, 256, 0)
    parts = _scatter_call(msg, dst2, zrows)
    h, _ = _node_call(h, rootp, parts, inv, biasr, gammar, betar, 128, 256, False)

    out = _pool_call(h, bs3, wpp, bpp)
    return out[:, :_NT]


# speedup vs baseline: 2.7736x; 1.0287x over previous
import functools

import jax
import jax.numpy as jnp
from jax import lax
from jax.experimental import pallas as pl
from jax.experimental.pallas import tpu as pltpu
from jax.experimental.pallas import tpu_sc as plsc

_N = 5000
_E = 10000
_NG = 256
_NT = 100

_NC = 2
_NS = 16
_NW = _NC * _NS
_CH = 80
_NCHUNK = 4
_TILE_E = _CH * _NCHUNK
_EP = _NW * _TILE_E
_NP = _NS * _TILE_E
_EB = 512
_NB = 256


def _sc_mesh():
    return plsc.VectorSubcoreMesh(core_axis_name="c", subcore_axis_name="s")


def _gather_call(h, src3):
    cinp = 128

    @functools.partial(
        pl.kernel,
        out_type=jax.ShapeDtypeStruct((_EP, cinp), jnp.float32),
        mesh=_sc_mesh(),
        scratch_types=[
            pltpu.VMEM((_NCHUNK, _CH), jnp.int32),
            pltpu.VMEM((_CH, cinp), jnp.float32),
            pltpu.SemaphoreType.DMA,
        ],
    )
    def k(h_hbm, src_hbm, out_hbm, idx_v, row_v, sem):
        c = lax.axis_index("c")
        s = lax.axis_index("s")
        wid = s * _NC + c
        pltpu.sync_copy(src_hbm.at[wid], idx_v)
        for j in range(_NCHUNK):
            pltpu.async_copy(h_hbm.at[idx_v.at[j]], row_v, sem).wait()
            pltpu.sync_copy(row_v, out_hbm.at[pl.ds(wid * _TILE_E + j * _CH, _CH)])

    return k(h, src3)


_EC = _EP // _NC
_CS = 16
_MCH = 1024


def _scatter_call(msgt, dst2, zrows):
    @functools.partial(
        pl.kernel,
        out_type=jax.ShapeDtypeStruct((2 * 256, _NP), jnp.float32),
        mesh=_sc_mesh(),
        compiler_params=pltpu.CompilerParams(needs_layout_passes=False),
        scratch_types=[
            pltpu.VMEM((_EC,), jnp.int32),
            pltpu.VMEM((_CS, _MCH), jnp.float32),
            pltpu.VMEM((_CS, _NP), jnp.float32),
        ],
    )
    def k(msg_hbm, dst_hbm, zero_hbm, out_hbm, dstv, mbuf, table):
        c = lax.axis_index("c")
        s = lax.axis_index("s")
        rows = lax.iota(jnp.int32, 16)
        for t in range(_NP // 512):
            pltpu.sync_copy(zero_hbm, table.at[:, pl.ds(t * 512, 512)])
        pltpu.sync_copy(dst_hbm.at[c], dstv)
        for t in range(_EC // _MCH):
            pltpu.sync_copy(
                msg_hbm.at[pl.ds(s * _CS, _CS), pl.ds(c * _EC + t * _MCH, _MCH)],
                mbuf,
            )

            def grp(i, _):
                d16 = dstv[pl.ds(t * _MCH + i * 16, 16)]
                for r in range(16):
                    vals = mbuf[r, pl.ds(i * 16, 16)]
                    rr = jnp.full((16,), r, jnp.int32)
                    plsc.addupdate_scatter(table, [rr, d16], vals)
                return _

            lax.fori_loop(0, _MCH // 16, grp, jnp.int32(0))
        pltpu.sync_copy(table, out_hbm.at[pl.ds(c * 256 + s * _CS, _CS)])

    return k(msgt, dst2, zrows)


def _msg_call(eap, hsrc, w1p, b1r, w2r, b2r, cinp, cout, ones_cols):
    nk = 32
    wtot = 256

    def body(ea_ref, hs_ref, w1_ref, b1_ref, w2_ref, b2_ref, out_ref):
        eh = jnp.maximum(
            jnp.dot(ea_ref[...], w1_ref[...], preferred_element_type=jnp.float32)
            + b1_ref[...],
            0.0,
        )
        hs = hs_ref[...][:, :cinp]
        q = jnp.concatenate([eh[:, k : k + 1] * hs for k in range(nk)], axis=1)
        msg = jnp.dot(
            q.astype(jnp.bfloat16), w2_ref[...], preferred_element_type=jnp.float32
        ) + jnp.dot(hs, b2_ref[...], preferred_element_type=jnp.float32)
        pieces = [msg]
        if ones_cols:
            pieces.append(jnp.ones((msg.shape[0], ones_cols), jnp.float32))
        pad = wtot - cout - ones_cols
        if pad:
            pieces.append(jnp.zeros((msg.shape[0], pad), jnp.float32))
        full = jnp.concatenate(pieces, axis=1) if len(pieces) > 1 else msg
        out_ref[...] = full.T

    return pl.pallas_call(
        body,
        grid=(_EP // _EB,),
        in_specs=[
            pl.BlockSpec((_EB, 8), lambda i: (i, 0)),
            pl.BlockSpec((_EB, 128), lambda i: (i, 0)),
            pl.BlockSpec((8, 32), lambda i: (0, 0)),
            pl.BlockSpec((1, 32), lambda i: (0, 0)),
            pl.BlockSpec((nk * cinp, cout), lambda i: (0, 0)),
            pl.BlockSpec((cinp, cout), lambda i: (0, 0)),
        ],
        out_specs=pl.BlockSpec((wtot, _EB), lambda i: (0, i)),
        out_shape=jax.ShapeDtypeStruct((wtot, _EP), jnp.float32),
    )(eap, hsrc, w1p, b1r, w2r, b2r)


def _node_call(h, rootp, parts, inv_or_cnt, biasr, gammar, betar, cinp, cout, first):
    wout = max(cout, 128)
    nblk = _NP // _NB

    def body(h_ref, root_ref, p0_ref, p1_ref, cv_ref, bias_ref,
             g_ref, beta_ref, out_ref, inv_ref):
        p0t = p0_ref[...]
        p1t = p1_ref[...]
        psum = (p0t[:cout, :] + p1t[:cout, :]).T
        if first:
            cntt = p0t[cout : cout + 16, :] + p1t[cout : cout + 16, :]
            cnt = cntt.T
            inv = 1.0 / jnp.maximum(cnt[:, :1], 1.0)
            inv_ref[...] = jnp.broadcast_to(inv, (_NB, 16))
        else:
            inv = cv_ref[...][:, :1]
        agg = psum * inv
        y = (
            jnp.dot(h_ref[...], root_ref[...], preferred_element_type=jnp.float32)
            + agg
            + bias_ref[...]
        )
        hv = jnp.maximum(y * g_ref[...] + beta_ref[...], 0.0)
        if wout > cout:
            hv = jnp.concatenate(
                [hv, jnp.zeros((_NB, wout - cout), jnp.float32)], axis=1
            )
        out_ref[...] = hv

    in_specs = [
        pl.BlockSpec((_NB, 128), lambda i: (i, 0)),
        pl.BlockSpec((128, cout), lambda i: (0, 0)),
        pl.BlockSpec((256, _NB), lambda i: (0, i)),
        pl.BlockSpec((256, _NB), lambda i: (1, i)),
        pl.BlockSpec((_NB, 16), lambda i: (i, 0)),
        pl.BlockSpec((1, cout), lambda i: (0, 0)),
        pl.BlockSpec((1, cout), lambda i: (0, 0)),
        pl.BlockSpec((1, cout), lambda i: (0, 0)),
    ]
    inv_in = jnp.zeros((_NP, 16), jnp.float32) if first else inv_or_cnt
    out = pl.pallas_call(
        body,
        grid=(nblk,),
        in_specs=in_specs,
        out_specs=[
            pl.BlockSpec((_NB, wout), lambda i: (i, 0)),
            pl.BlockSpec((_NB, 16), lambda i: (i, 0)),
        ],
        out_shape=[
            jax.ShapeDtypeStruct((_NP, wout), jnp.float32),
            jax.ShapeDtypeStruct((_NP, 16), jnp.float32),
        ],
    )(h, rootp, parts, parts, inv_in, biasr, gammar, betar)
    return out


def _pool_call(h3, bs3, wpp, bpp):
    nblk = _NP // _NB

    def body(h_ref, bs_ref, wp_ref, bp_ref, out_ref, acc, pcnt):
        i = pl.program_id(0)

        @pl.when(i == 0)
        def _init():
            acc[...] = jnp.zeros_like(acc)
            pcnt[...] = jnp.zeros_like(pcnt)

        seg = lax.broadcasted_iota(jnp.int32, (_NG, _NB), 0)
        bs = bs_ref[0]
        oh = (seg == bs).astype(jnp.float32)
        acc[...] += jnp.dot(oh, h_ref[...], preferred_element_type=jnp.float32)
        pcnt[...] += jnp.broadcast_to(
            jnp.sum(oh, axis=1, keepdims=True), (_NG, 128)
        )

        @pl.when(i == nblk - 1)
        def _fin():
            pooled = acc[...] * (1.0 / jnp.maximum(pcnt[...][:, :1], 1.0))
            o = jnp.dot(
                pooled, wp_ref[...], preferred_element_type=jnp.float32
            ) + bp_ref[...]
            out_ref[...] = jnp.where(o > 0, o, 0.1 * o)

    return pl.pallas_call(
        body,
        grid=(nblk,),
        in_specs=[
            pl.BlockSpec((_NB, 256), lambda i: (i, 0)),
            pl.BlockSpec((1, 1, _NB), lambda i: (i, 0, 0)),
            pl.BlockSpec((256, 128), lambda i: (0, 0)),
            pl.BlockSpec((1, 128), lambda i: (0, 0)),
        ],
        out_specs=pl.BlockSpec((_NG, 128), lambda i: (0, 0)),
        out_shape=jax.ShapeDtypeStruct((_NG, 128), jnp.float32),
        scratch_shapes=[
            pltpu.VMEM((_NG, 256), jnp.float32),
            pltpu.VMEM((_NG, 128), jnp.float32),
        ],
    )(h3, bs3, wpp, bpp)


def _prep_layer(p, cin, cinp, cout):
    w1p = jnp.zeros((8, 32), jnp.float32).at[:3].set(p["W1"])
    b1r = p["b1"].reshape(1, 32)
    w2 = p["W2"].reshape(32, cin, cout)
    w2r = (
        jnp.zeros((32, cinp, cout), jnp.float32)
        .at[:, :cin, :]
        .set(w2)
        .reshape(32 * cinp, cout)
        .astype(jnp.bfloat16)
    )
    b2r = jnp.zeros((cinp, cout), jnp.float32).at[:cin].set(
        p["b2"].reshape(cin, cout)
    )
    rootp = jnp.zeros((128, cout), jnp.float32).at[:cin].set(p["root"])
    biasr = p["bias"].reshape(1, cout)
    gammar = (p["gamma"] / jnp.sqrt(1.0 + 1e-5)).reshape(1, cout)
    betar = p["beta"].reshape(1, cout)
    return w1p, b1r, w2r, b2r, rootp, biasr, gammar, betar


def kernel(x, edge_index, edge_attr, batch_seg, params):
    f32 = jnp.float32
    src = edge_index[0]
    dst = edge_index[1]
    xp = jnp.zeros((_NP, 128), f32).at[:_N, :5].set(x)
    src3 = (
        jnp.zeros((_EP,), jnp.int32).at[:_E].set(src).reshape(_NW, _NCHUNK, _CH)
    )
    dst2 = jnp.full((_EP,), _N, jnp.int32).at[:_E].set(dst).reshape(_NC, _EC)
    eap = jnp.zeros((_EP, 8), f32).at[:_E, :3].set(edge_attr)
    bs3 = (
        jnp.full((_NP,), _NG + 8, jnp.int32)
        .at[:_N]
        .set(batch_seg)
        .reshape(_NP // _NB, 1, _NB)
    )
    zrows = jnp.zeros((_CS, 512), f32)
    l1 = _prep_layer(params["layer1"], 5, 16, 64)
    l2 = _prep_layer(params["layer2"], 64, 64, 128)
    l3 = _prep_layer(params["layer3"], 128, 128, 256)
    wpp = jnp.zeros((256, 128), f32).at[:, :_NT].set(params["mlp_W"])
    bpp = jnp.zeros((1, 128), f32).at[0, :_NT].set(params["mlp_b"])

    w1p, b1r, w2r, b2r, rootp, biasr, gammar, betar = l1
    hs = _gather_call(xp, src3)
    msg = _msg_call(eap, hs, w1p, b1r, w2r, b2r, 16, 64, 16)
    parts = _scatter_call(msg, dst2, zrows)
    h, inv = _node_call(xp, rootp, parts, None, biasr, gammar, betar, 16, 64, True)

    w1p, b1r, w2r, b2r, rootp, biasr, gammar, betar = l2
    hs = _gather_call(h, src3)
    msg = _msg_call(eap, hs, w1p, b1r, w2r, b2r, 64, 128, 0)
    parts = _scatter_call(msg, dst2, zrows)
    h, _ = _node_call(h, rootp, parts, inv, biasr, gammar, betar, 64, 128, False)

    w1p, b1r, w2r, b2r, rootp, biasr, gammar, betar = l3
    hs = _gather_call(h, src3)
    msg = _msg_call(eap, hs, w1p, b1r, w2r, b2r, 128, 256, 0)
    parts = _scatter_call(msg, dst2, zrows)
    h, _ = _node_call(h, rootp, parts, inv, biasr, gammar, betar, 128, 256, False)

    out = _pool_call(h, bs3, wpp, bpp)
    return out[:, :_NT]


# speedup vs baseline: 2.9662x; 1.0695x over previous
import functools

import jax
import jax.numpy as jnp
from jax import lax
from jax.experimental import pallas as pl
from jax.experimental.pallas import tpu as pltpu
from jax.experimental.pallas import tpu_sc as plsc

_N = 5000
_E = 10000
_NG = 256
_NT = 100

_NC = 2
_NS = 16
_NW = _NC * _NS
_CH = 80
_NCHUNK = 4
_TILE_E = _CH * _NCHUNK
_EP = _NW * _TILE_E
_NP = _NS * _TILE_E
_EB = 512
_NB = 256


def _sc_mesh():
    return plsc.VectorSubcoreMesh(core_axis_name="c", subcore_axis_name="s")


def _gather_call(h, src3):
    cinp = 128

    @functools.partial(
        pl.kernel,
        out_type=jax.ShapeDtypeStruct((_EP, cinp), jnp.float32),
        mesh=_sc_mesh(),
        scratch_types=[
            pltpu.VMEM((_NCHUNK, _CH), jnp.int32),
            pltpu.VMEM((_CH, cinp), jnp.float32),
            pltpu.SemaphoreType.DMA,
        ],
    )
    def k(h_hbm, src_hbm, out_hbm, idx_v, row_v, sem):
        c = lax.axis_index("c")
        s = lax.axis_index("s")
        wid = s * _NC + c
        pltpu.sync_copy(src_hbm.at[wid], idx_v)
        for j in range(_NCHUNK):
            pltpu.async_copy(h_hbm.at[idx_v.at[j]], row_v, sem).wait()
            pltpu.sync_copy(row_v, out_hbm.at[pl.ds(wid * _TILE_E + j * _CH, _CH)])

    return k(h, src3)


_EC = _EP // _NC
_CS = 16
_MCH = 1024


def _scatter_call(msgt, dst2, zrows):
    @functools.partial(
        pl.kernel,
        out_type=jax.ShapeDtypeStruct((2 * 256, _NP), jnp.float32),
        mesh=_sc_mesh(),
        compiler_params=pltpu.CompilerParams(needs_layout_passes=False),
        scratch_types=[
            pltpu.VMEM((_EC,), jnp.int32),
            pltpu.VMEM((_CS, _MCH), jnp.float32),
            pltpu.VMEM((_CS, _NP), jnp.float32),
        ],
    )
    def k(msg_hbm, dst_hbm, zero_hbm, out_hbm, dstv, mbuf, table):
        c = lax.axis_index("c")
        s = lax.axis_index("s")
        rows = lax.iota(jnp.int32, 16)
        for t in range(_NP // 512):
            pltpu.sync_copy(zero_hbm, table.at[:, pl.ds(t * 512, 512)])
        pltpu.sync_copy(dst_hbm.at[c], dstv)
        for t in range(_EC // _MCH):
            pltpu.sync_copy(
                msg_hbm.at[pl.ds(s * _CS, _CS), pl.ds(c * _EC + t * _MCH, _MCH)],
                mbuf,
            )

            def grp(i, _):
                d16 = dstv[pl.ds(t * _MCH + i * 16, 16)]
                for r in range(16):
                    vals = mbuf[r, pl.ds(i * 16, 16)]
                    rr = jnp.full((16,), r, jnp.int32)
                    plsc.addupdate_scatter(table, [rr, d16], vals)
                return _

            lax.fori_loop(0, _MCH // 16, grp, jnp.int32(0))
        pltpu.sync_copy(table, out_hbm.at[pl.ds(c * 256 + s * _CS, _CS)])

    return k(msgt, dst2, zrows)


def _msg_call(eap, hsrc, w1p, b1r, w2r, b2r, smat, tmat, cinp, cout, ones_cols):
    nk = 32
    wtot = 256

    def body(ea_ref, hs_ref, w1_ref, b1_ref, w2_ref, b2_ref, s_ref, t_ref,
             out_ref):
        eh = jnp.maximum(
            jnp.dot(ea_ref[...], w1_ref[...], preferred_element_type=jnp.float32)
            + b1_ref[...],
            0.0,
        )
        hs = hs_ref[...][:, :cinp]
        ehb = jnp.dot(
            eh.astype(jnp.bfloat16), s_ref[...],
            preferred_element_type=jnp.float32,
        )
        hst = jnp.dot(
            hs.astype(jnp.bfloat16), t_ref[...],
            preferred_element_type=jnp.float32,
        )
        q = (ehb * hst).astype(jnp.bfloat16)
        msg = jnp.dot(
            q, w2_ref[...], preferred_element_type=jnp.float32
        ) + jnp.dot(hs, b2_ref[...], preferred_element_type=jnp.float32)
        pieces = [msg]
        if ones_cols:
            pieces.append(jnp.ones((msg.shape[0], ones_cols), jnp.float32))
        pad = wtot - cout - ones_cols
        if pad:
            pieces.append(jnp.zeros((msg.shape[0], pad), jnp.float32))
        full = jnp.concatenate(pieces, axis=1) if len(pieces) > 1 else msg
        out_ref[...] = full.T

    return pl.pallas_call(
        body,
        grid=(_EP // _EB,),
        in_specs=[
            pl.BlockSpec((_EB, 8), lambda i: (i, 0)),
            pl.BlockSpec((_EB, 128), lambda i: (i, 0)),
            pl.BlockSpec((8, 32), lambda i: (0, 0)),
            pl.BlockSpec((1, 32), lambda i: (0, 0)),
            pl.BlockSpec((nk * cinp, cout), lambda i: (0, 0)),
            pl.BlockSpec((cinp, cout), lambda i: (0, 0)),
            pl.BlockSpec((nk, nk * cinp), lambda i: (0, 0)),
            pl.BlockSpec((cinp, nk * cinp), lambda i: (0, 0)),
        ],
        out_specs=pl.BlockSpec((wtot, _EB), lambda i: (0, i)),
        out_shape=jax.ShapeDtypeStruct((wtot, _EP), jnp.float32),
    )(eap, hsrc, w1p, b1r, w2r, b2r, smat, tmat)


def _node_call(h, rootp, parts, inv_or_cnt, biasr, gammar, betar, cinp, cout, first):
    wout = max(cout, 128)
    nblk = _NP // _NB

    def body(h_ref, root_ref, p0_ref, p1_ref, cv_ref, bias_ref,
             g_ref, beta_ref, out_ref, inv_ref):
        p0t = p0_ref[...]
        p1t = p1_ref[...]
        psum = (p0t[:cout, :] + p1t[:cout, :]).T
        if first:
            cntt = p0t[cout : cout + 16, :] + p1t[cout : cout + 16, :]
            cnt = cntt.T
            inv = 1.0 / jnp.maximum(cnt[:, :1], 1.0)
            inv_ref[...] = jnp.broadcast_to(inv, (_NB, 16))
        else:
            inv = cv_ref[...][:, :1]
        agg = psum * inv
        y = (
            jnp.dot(h_ref[...], root_ref[...], preferred_element_type=jnp.float32)
            + agg
            + bias_ref[...]
        )
        hv = jnp.maximum(y * g_ref[...] + beta_ref[...], 0.0)
        if wout > cout:
            hv = jnp.concatenate(
                [hv, jnp.zeros((_NB, wout - cout), jnp.float32)], axis=1
            )
        out_ref[...] = hv

    in_specs = [
        pl.BlockSpec((_NB, 128), lambda i: (i, 0)),
        pl.BlockSpec((128, cout), lambda i: (0, 0)),
        pl.BlockSpec((256, _NB), lambda i: (0, i)),
        pl.BlockSpec((256, _NB), lambda i: (1, i)),
        pl.BlockSpec((_NB, 16), lambda i: (i, 0)),
        pl.BlockSpec((1, cout), lambda i: (0, 0)),
        pl.BlockSpec((1, cout), lambda i: (0, 0)),
        pl.BlockSpec((1, cout), lambda i: (0, 0)),
    ]
    inv_in = jnp.zeros((_NP, 16), jnp.float32) if first else inv_or_cnt
    out = pl.pallas_call(
        body,
        grid=(nblk,),
        in_specs=in_specs,
        out_specs=[
            pl.BlockSpec((_NB, wout), lambda i: (i, 0)),
            pl.BlockSpec((_NB, 16), lambda i: (i, 0)),
        ],
        out_shape=[
            jax.ShapeDtypeStruct((_NP, wout), jnp.float32),
            jax.ShapeDtypeStruct((_NP, 16), jnp.float32),
        ],
    )(h, rootp, parts, parts, inv_in, biasr, gammar, betar)
    return out


def _pool_call(h3, bs3, wpp, bpp):
    nblk = _NP // _NB

    def body(h_ref, bs_ref, wp_ref, bp_ref, out_ref, acc, pcnt):
        i = pl.program_id(0)

        @pl.when(i == 0)
        def _init():
            acc[...] = jnp.zeros_like(acc)
            pcnt[...] = jnp.zeros_like(pcnt)

        seg = lax.broadcasted_iota(jnp.int32, (_NG, _NB), 0)
        bs = bs_ref[0]
        oh = (seg == bs).astype(jnp.float32)
        acc[...] += jnp.dot(oh, h_ref[...], preferred_element_type=jnp.float32)
        pcnt[...] += jnp.broadcast_to(
            jnp.sum(oh, axis=1, keepdims=True), (_NG, 128)
        )

        @pl.when(i == nblk - 1)
        def _fin():
            pooled = acc[...] * (1.0 / jnp.maximum(pcnt[...][:, :1], 1.0))
            o = jnp.dot(
                pooled, wp_ref[...], preferred_element_type=jnp.float32
            ) + bp_ref[...]
            out_ref[...] = jnp.where(o > 0, o, 0.1 * o)

    return pl.pallas_call(
        body,
        grid=(nblk,),
        in_specs=[
            pl.BlockSpec((_NB, 256), lambda i: (i, 0)),
            pl.BlockSpec((1, 1, _NB), lambda i: (i, 0, 0)),
            pl.BlockSpec((256, 128), lambda i: (0, 0)),
            pl.BlockSpec((1, 128), lambda i: (0, 0)),
        ],
        out_specs=pl.BlockSpec((_NG, 128), lambda i: (0, 0)),
        out_shape=jax.ShapeDtypeStruct((_NG, 128), jnp.float32),
        scratch_shapes=[
            pltpu.VMEM((_NG, 256), jnp.float32),
            pltpu.VMEM((_NG, 128), jnp.float32),
        ],
    )(h3, bs3, wpp, bpp)


def _prep_layer(p, cin, cinp, cout):
    w1p = jnp.zeros((8, 32), jnp.float32).at[:3].set(p["W1"])
    b1r = p["b1"].reshape(1, 32)
    w2 = p["W2"].reshape(32, cin, cout)
    w2r = (
        jnp.zeros((32, cinp, cout), jnp.float32)
        .at[:, :cin, :]
        .set(w2)
        .reshape(32 * cinp, cout)
        .astype(jnp.bfloat16)
    )
    b2r = jnp.zeros((cinp, cout), jnp.float32).at[:cin].set(
        p["b2"].reshape(cin, cout)
    )
    kk = jnp.arange(32 * cinp)
    smat = (kk[None, :] // cinp == jnp.arange(32)[:, None]).astype(jnp.bfloat16)
    tmat = (kk[None, :] % cinp == jnp.arange(cinp)[:, None]).astype(jnp.bfloat16)
    rootp = jnp.zeros((128, cout), jnp.float32).at[:cin].set(p["root"])
    biasr = p["bias"].reshape(1, cout)
    gammar = (p["gamma"] / jnp.sqrt(1.0 + 1e-5)).reshape(1, cout)
    betar = p["beta"].reshape(1, cout)
    return w1p, b1r, w2r, b2r, smat, tmat, rootp, biasr, gammar, betar


def kernel(x, edge_index, edge_attr, batch_seg, params):
    f32 = jnp.float32
    src = edge_index[0]
    dst = edge_index[1]
    xp = jnp.zeros((_NP, 128), f32).at[:_N, :5].set(x)
    src3 = (
        jnp.zeros((_EP,), jnp.int32).at[:_E].set(src).reshape(_NW, _NCHUNK, _CH)
    )
    dst2 = jnp.full((_EP,), _N, jnp.int32).at[:_E].set(dst).reshape(_NC, _EC)
    eap = jnp.zeros((_EP, 8), f32).at[:_E, :3].set(edge_attr)
    bs3 = (
        jnp.full((_NP,), _NG + 8, jnp.int32)
        .at[:_N]
        .set(batch_seg)
        .reshape(_NP // _NB, 1, _NB)
    )
    zrows = jnp.zeros((_CS, 512), f32)
    l1 = _prep_layer(params["layer1"], 5, 16, 64)
    l2 = _prep_layer(params["layer2"], 64, 64, 128)
    l3 = _prep_layer(params["layer3"], 128, 128, 256)
    wpp = jnp.zeros((256, 128), f32).at[:, :_NT].set(params["mlp_W"])
    bpp = jnp.zeros((1, 128), f32).at[0, :_NT].set(params["mlp_b"])

    w1p, b1r, w2r, b2r, smat, tmat, rootp, biasr, gammar, betar = l1
    hs = _gather_call(xp, src3)
    msg = _msg_call(eap, hs, w1p, b1r, w2r, b2r, smat, tmat, 16, 64, 16)
    parts = _scatter_call(msg, dst2, zrows)
    h, inv = _node_call(xp, rootp, parts, None, biasr, gammar, betar, 16, 64, True)

    w1p, b1r, w2r, b2r, smat, tmat, rootp, biasr, gammar, betar = l2
    hs = _gather_call(h, src3)
    msg = _msg_call(eap, hs, w1p, b1r, w2r, b2r, smat, tmat, 64, 128, 0)
    parts = _scatter_call(msg, dst2, zrows)
    h, _ = _node_call(h, rootp, parts, inv, biasr, gammar, betar, 64, 128, False)

    w1p, b1r, w2r, b2r, smat, tmat, rootp, biasr, gammar, betar = l3
    hs = _gather_call(h, src3)
    msg = _msg_call(eap, hs, w1p, b1r, w2r, b2r, smat, tmat, 128, 256, 0)
    parts = _scatter_call(msg, dst2, zrows)
    h, _ = _node_call(h, rootp, parts, inv, biasr, gammar, betar, 128, 256, False)

    out = _pool_call(h, bs3, wpp, bpp)
    return out[:, :_NT]


# speedup vs baseline: 3.4626x; 1.1673x over previous
import functools

import jax
import jax.numpy as jnp
from jax import lax
from jax.experimental import pallas as pl
from jax.experimental.pallas import tpu as pltpu
from jax.experimental.pallas import tpu_sc as plsc

_N = 5000
_E = 10000
_NG = 256
_NT = 100

_NC = 2
_NS = 16
_NW = _NC * _NS
_CH = 80
_NCHUNK = 4
_TILE_E = _CH * _NCHUNK
_EP = _NW * _TILE_E
_NP = _NS * _TILE_E
_EB = 512
_NB = 256


def _sc_mesh():
    return plsc.VectorSubcoreMesh(core_axis_name="c", subcore_axis_name="s")


def _gather_call(h, src3):
    cinp = 128

    @functools.partial(
        pl.kernel,
        out_type=jax.ShapeDtypeStruct((_EP, cinp), jnp.float32),
        mesh=_sc_mesh(),
        scratch_types=[
            pltpu.VMEM((_NCHUNK, _CH), jnp.int32),
            [pltpu.VMEM((_CH, cinp), jnp.float32) for _ in range(_NCHUNK)],
            [pltpu.SemaphoreType.DMA for _ in range(_NCHUNK)],
            [pltpu.SemaphoreType.DMA for _ in range(_NCHUNK)],
        ],
    )
    def k(h_hbm, src_hbm, out_hbm, idx_v, rows, gsems, wsems):
        c = lax.axis_index("c")
        s = lax.axis_index("s")
        wid = s * _NC + c
        pltpu.sync_copy(src_hbm.at[wid], idx_v)
        gcps = [
            pltpu.async_copy(h_hbm.at[idx_v.at[j]], rows[j], gsems[j])
            for j in range(_NCHUNK)
        ]
        wcps = []
        for j in range(_NCHUNK):
            gcps[j].wait()
            wcps.append(
                pltpu.async_copy(
                    rows[j],
                    out_hbm.at[pl.ds(wid * _TILE_E + j * _CH, _CH)],
                    wsems[j],
                )
            )
        for w in wcps:
            w.wait()

    return k(h, src3)


_EC = _EP // _NC
_CS = 16
_MCH = 1024


def _scatter_call(msgt, dst2, zrows):
    @functools.partial(
        pl.kernel,
        out_type=jax.ShapeDtypeStruct((2 * 256, _NP), jnp.float32),
        mesh=_sc_mesh(),
        compiler_params=pltpu.CompilerParams(needs_layout_passes=False),
        scratch_types=[
            pltpu.VMEM((_EC,), jnp.int32),
            [pltpu.VMEM((_CS, _MCH), jnp.float32) for _ in range(2)],
            pltpu.VMEM((_CS, _NP), jnp.float32),
            [pltpu.SemaphoreType.DMA for _ in range(4)],
        ],
    )
    def k(msg_hbm, dst_hbm, zero_hbm, out_hbm, dstv, mbufs, table, sems):
        c = lax.axis_index("c")
        s = lax.axis_index("s")
        nch = _EC // _MCH

        def chunk_cp(t, buf, sem):
            return pltpu.async_copy(
                msg_hbm.at[
                    pl.ds(s * _CS, _CS), pl.ds(c * _EC + t * _MCH, _MCH)
                ],
                buf,
                sem,
            )

        zc = pltpu.async_copy(zero_hbm, table, sems[2])
        dc = pltpu.async_copy(dst_hbm.at[c], dstv, sems[3])
        cps = [chunk_cp(0, mbufs[0], sems[0])]
        dc.wait()
        zc.wait()
        for t in range(nch):
            if t + 1 < nch:
                cps.append(chunk_cp(t + 1, mbufs[(t + 1) % 2], sems[(t + 1) % 2]))
            cps[t].wait()
            mbuf = mbufs[t % 2]

            def grp(i, _):
                d16 = dstv[pl.ds(t * _MCH + i * 16, 16)]
                for r in range(16):
                    vals = mbuf[r, pl.ds(i * 16, 16)]
                    rr = jnp.full((16,), r, jnp.int32)
                    plsc.addupdate_scatter(table, [rr, d16], vals)
                return _

            lax.fori_loop(0, _MCH // 16, grp, jnp.int32(0))
        pltpu.sync_copy(table, out_hbm.at[pl.ds(c * 256 + s * _CS, _CS)])

    return k(msgt, dst2, zrows)


def _msg_call(eap, hsrc, w1p, b1r, w2r, b2r, smat, tmat, cinp, cout, ones_cols):
    nk = 32
    wtot = 256

    def body(ea_ref, hs_ref, w1_ref, b1_ref, w2_ref, b2_ref, s_ref, t_ref,
             out_ref):
        eh = jnp.maximum(
            jnp.dot(ea_ref[...], w1_ref[...], preferred_element_type=jnp.float32)
            + b1_ref[...],
            0.0,
        )
        hs = hs_ref[...][:, :cinp]
        ehb = jnp.dot(
            eh.astype(jnp.bfloat16), s_ref[...],
            preferred_element_type=jnp.float32,
        )
        hst = jnp.dot(
            hs.astype(jnp.bfloat16), t_ref[...],
            preferred_element_type=jnp.float32,
        )
        q = (ehb * hst).astype(jnp.bfloat16)
        msg = jnp.dot(
            q, w2_ref[...], preferred_element_type=jnp.float32
        ) + jnp.dot(hs, b2_ref[...], preferred_element_type=jnp.float32)
        pieces = [msg]
        if ones_cols:
            pieces.append(jnp.ones((msg.shape[0], ones_cols), jnp.float32))
        pad = wtot - cout - ones_cols
        if pad:
            pieces.append(jnp.zeros((msg.shape[0], pad), jnp.float32))
        full = jnp.concatenate(pieces, axis=1) if len(pieces) > 1 else msg
        out_ref[...] = full.T

    return pl.pallas_call(
        body,
        grid=(_EP // _EB,),
        in_specs=[
            pl.BlockSpec((_EB, 8), lambda i: (i, 0)),
            pl.BlockSpec((_EB, 128), lambda i: (i, 0)),
            pl.BlockSpec((8, 32), lambda i: (0, 0)),
            pl.BlockSpec((1, 32), lambda i: (0, 0)),
            pl.BlockSpec((nk * cinp, cout), lambda i: (0, 0)),
            pl.BlockSpec((cinp, cout), lambda i: (0, 0)),
            pl.BlockSpec((nk, nk * cinp), lambda i: (0, 0)),
            pl.BlockSpec((cinp, nk * cinp), lambda i: (0, 0)),
        ],
        out_specs=pl.BlockSpec((wtot, _EB), lambda i: (0, i)),
        out_shape=jax.ShapeDtypeStruct((wtot, _EP), jnp.float32),
    )(eap, hsrc, w1p, b1r, w2r, b2r, smat, tmat)


def _node_call(h, rootp, parts, inv_or_cnt, biasr, gammar, betar, cinp, cout, first):
    wout = max(cout, 128)
    nblk = _NP // _NB

    def body(h_ref, root_ref, p0_ref, p1_ref, cv_ref, bias_ref,
             g_ref, beta_ref, out_ref, inv_ref):
        p0t = p0_ref[...]
        p1t = p1_ref[...]
        psum = (p0t[:cout, :] + p1t[:cout, :]).T
        if first:
            cntt = p0t[cout : cout + 16, :] + p1t[cout : cout + 16, :]
            cnt = cntt.T
            inv = 1.0 / jnp.maximum(cnt[:, :1], 1.0)
            inv_ref[...] = jnp.broadcast_to(inv, (_NB, 16))
        else:
            inv = cv_ref[...][:, :1]
        agg = psum * inv
        y = (
            jnp.dot(h_ref[...], root_ref[...], preferred_element_type=jnp.float32)
            + agg
            + bias_ref[...]
        )
        hv = jnp.maximum(y * g_ref[...] + beta_ref[...], 0.0)
        if wout > cout:
            hv = jnp.concatenate(
                [hv, jnp.zeros((_NB, wout - cout), jnp.float32)], axis=1
            )
        out_ref[...] = hv

    in_specs = [
        pl.BlockSpec((_NB, 128), lambda i: (i, 0)),
        pl.BlockSpec((128, cout), lambda i: (0, 0)),
        pl.BlockSpec((256, _NB), lambda i: (0, i)),
        pl.BlockSpec((256, _NB), lambda i: (1, i)),
        pl.BlockSpec((_NB, 16), lambda i: (i, 0)),
        pl.BlockSpec((1, cout), lambda i: (0, 0)),
        pl.BlockSpec((1, cout), lambda i: (0, 0)),
        pl.BlockSpec((1, cout), lambda i: (0, 0)),
    ]
    inv_in = jnp.zeros((_NP, 16), jnp.float32) if first else inv_or_cnt
    out = pl.pallas_call(
        body,
        grid=(nblk,),
        in_specs=in_specs,
        out_specs=[
            pl.BlockSpec((_NB, wout), lambda i: (i, 0)),
            pl.BlockSpec((_NB, 16), lambda i: (i, 0)),
        ],
        out_shape=[
            jax.ShapeDtypeStruct((_NP, wout), jnp.float32),
            jax.ShapeDtypeStruct((_NP, 16), jnp.float32),
        ],
    )(h, rootp, parts, parts, inv_in, biasr, gammar, betar)
    return out


def _pool_call(h3, bs3, wpp, bpp):
    nblk = _NP // _NB

    def body(h_ref, bs_ref, wp_ref, bp_ref, out_ref, acc, pcnt):
        i = pl.program_id(0)

        @pl.when(i == 0)
        def _init():
            acc[...] = jnp.zeros_like(acc)
            pcnt[...] = jnp.zeros_like(pcnt)

        seg = lax.broadcasted_iota(jnp.int32, (_NG, _NB), 0)
        bs = bs_ref[0]
        oh = (seg == bs).astype(jnp.float32)
        acc[...] += jnp.dot(oh, h_ref[...], preferred_element_type=jnp.float32)
        pcnt[...] += jnp.broadcast_to(
            jnp.sum(oh, axis=1, keepdims=True), (_NG, 128)
        )

        @pl.when(i == nblk - 1)
        def _fin():
            pooled = acc[...] * (1.0 / jnp.maximum(pcnt[...][:, :1], 1.0))
            o = jnp.dot(
                pooled, wp_ref[...], preferred_element_type=jnp.float32
            ) + bp_ref[...]
            out_ref[...] = jnp.where(o > 0, o, 0.1 * o)

    return pl.pallas_call(
        body,
        grid=(nblk,),
        in_specs=[
            pl.BlockSpec((_NB, 256), lambda i: (i, 0)),
            pl.BlockSpec((1, 1, _NB), lambda i: (i, 0, 0)),
            pl.BlockSpec((256, 128), lambda i: (0, 0)),
            pl.BlockSpec((1, 128), lambda i: (0, 0)),
        ],
        out_specs=pl.BlockSpec((_NG, 128), lambda i: (0, 0)),
        out_shape=jax.ShapeDtypeStruct((_NG, 128), jnp.float32),
        scratch_shapes=[
            pltpu.VMEM((_NG, 256), jnp.float32),
            pltpu.VMEM((_NG, 128), jnp.float32),
        ],
    )(h3, bs3, wpp, bpp)


def _prep_layer(p, cin, cinp, cout):
    w1p = jnp.zeros((8, 32), jnp.float32).at[:3].set(p["W1"])
    b1r = p["b1"].reshape(1, 32)
    w2 = p["W2"].reshape(32, cin, cout)
    w2r = (
        jnp.zeros((32, cinp, cout), jnp.float32)
        .at[:, :cin, :]
        .set(w2)
        .reshape(32 * cinp, cout)
        .astype(jnp.bfloat16)
    )
    b2r = jnp.zeros((cinp, cout), jnp.float32).at[:cin].set(
        p["b2"].reshape(cin, cout)
    )
    kk = jnp.arange(32 * cinp)
    smat = (kk[None, :] // cinp == jnp.arange(32)[:, None]).astype(jnp.bfloat16)
    tmat = (kk[None, :] % cinp == jnp.arange(cinp)[:, None]).astype(jnp.bfloat16)
    rootp = jnp.zeros((128, cout), jnp.float32).at[:cin].set(p["root"])
    biasr = p["bias"].reshape(1, cout)
    gammar = (p["gamma"] / jnp.sqrt(1.0 + 1e-5)).reshape(1, cout)
    betar = p["beta"].reshape(1, cout)
    return w1p, b1r, w2r, b2r, smat, tmat, rootp, biasr, gammar, betar


def kernel(x, edge_index, edge_attr, batch_seg, params):
    f32 = jnp.float32
    src = edge_index[0]
    dst = edge_index[1]
    xp = jnp.zeros((_NP, 128), f32).at[:_N, :5].set(x)
    src3 = (
        jnp.zeros((_EP,), jnp.int32).at[:_E].set(src).reshape(_NW, _NCHUNK, _CH)
    )
    dst2 = jnp.full((_EP,), _N, jnp.int32).at[:_E].set(dst).reshape(_NC, _EC)
    eap = jnp.zeros((_EP, 8), f32).at[:_E, :3].set(edge_attr)
    bs3 = (
        jnp.full((_NP,), _NG + 8, jnp.int32)
        .at[:_N]
        .set(batch_seg)
        .reshape(_NP // _NB, 1, _NB)
    )
    zrows = jnp.zeros((_CS, _NP), f32)
    l1 = _prep_layer(params["layer1"], 5, 16, 64)
    l2 = _prep_layer(params["layer2"], 64, 64, 128)
    l3 = _prep_layer(params["layer3"], 128, 128, 256)
    wpp = jnp.zeros((256, 128), f32).at[:, :_NT].set(params["mlp_W"])
    bpp = jnp.zeros((1, 128), f32).at[0, :_NT].set(params["mlp_b"])

    w1p, b1r, w2r, b2r, smat, tmat, rootp, biasr, gammar, betar = l1
    hs = _gather_call(xp, src3)
    msg = _msg_call(eap, hs, w1p, b1r, w2r, b2r, smat, tmat, 16, 64, 16)
    parts = _scatter_call(msg, dst2, zrows)
    h, inv = _node_call(xp, rootp, parts, None, biasr, gammar, betar, 16, 64, True)

    w1p, b1r, w2r, b2r, smat, tmat, rootp, biasr, gammar, betar = l2
    hs = _gather_call(h, src3)
    msg = _msg_call(eap, hs, w1p, b1r, w2r, b2r, smat, tmat, 64, 128, 0)
    parts = _scatter_call(msg, dst2, zrows)
    h, _ = _node_call(h, rootp, parts, inv, biasr, gammar, betar, 64, 128, False)

    w1p, b1r, w2r, b2r, smat, tmat, rootp, biasr, gammar, betar = l3
    hs = _gather_call(h, src3)
    msg = _msg_call(eap, hs, w1p, b1r, w2r, b2r, smat, tmat, 128, 256, 0)
    parts = _scatter_call(msg, dst2, zrows)
    h, _ = _node_call(h, rootp, parts, inv, biasr, gammar, betar, 128, 256, False)

    out = _pool_call(h, bs3, wpp, bpp)
    return out[:, :_NT]


# speedup vs baseline: 3.5512x; 1.0256x over previous
import functools

import jax
import jax.numpy as jnp
from jax import lax
from jax.experimental import pallas as pl
from jax.experimental.pallas import tpu as pltpu
from jax.experimental.pallas import tpu_sc as plsc

_N = 5000
_E = 10000
_NG = 256
_NT = 100

_NC = 2
_NS = 16
_NW = _NC * _NS
_CH = 80
_NCHUNK = 4
_TILE_E = _CH * _NCHUNK
_EP = _NW * _TILE_E
_NP = _NS * _TILE_E
_EB = 512
_NB = 256


def _sc_mesh():
    return plsc.VectorSubcoreMesh(core_axis_name="c", subcore_axis_name="s")


def _gather_call(h, src3):
    cinp = 128

    @functools.partial(
        pl.kernel,
        out_type=jax.ShapeDtypeStruct((_EP, cinp), jnp.float32),
        mesh=_sc_mesh(),
        scratch_types=[
            pltpu.VMEM((_NCHUNK, _CH), jnp.int32),
            [pltpu.VMEM((_CH, cinp), jnp.float32) for _ in range(_NCHUNK)],
            [pltpu.SemaphoreType.DMA for _ in range(_NCHUNK)],
            [pltpu.SemaphoreType.DMA for _ in range(_NCHUNK)],
        ],
    )
    def k(h_hbm, src_hbm, out_hbm, idx_v, rows, gsems, wsems):
        c = lax.axis_index("c")
        s = lax.axis_index("s")
        wid = s * _NC + c
        pltpu.sync_copy(src_hbm.at[wid], idx_v)
        gcps = [
            pltpu.async_copy(h_hbm.at[idx_v.at[j]], rows[j], gsems[j])
            for j in range(_NCHUNK)
        ]
        wcps = []
        for j in range(_NCHUNK):
            gcps[j].wait()
            wcps.append(
                pltpu.async_copy(
                    rows[j],
                    out_hbm.at[pl.ds(wid * _TILE_E + j * _CH, _CH)],
                    wsems[j],
                )
            )
        for w in wcps:
            w.wait()

    return k(h, src3)


_EC = _EP // _NC
_CS = 16
_MCH = 1024


def _scatter_call(msgt, dst2, zrows):
    @functools.partial(
        pl.kernel,
        out_type=jax.ShapeDtypeStruct((2 * 256, _NP), jnp.float32),
        mesh=_sc_mesh(),
        compiler_params=pltpu.CompilerParams(needs_layout_passes=False),
        scratch_types=[
            pltpu.VMEM((_EC,), jnp.int32),
            [pltpu.VMEM((_CS, _MCH), jnp.float32) for _ in range(2)],
            pltpu.VMEM((_CS, _NP), jnp.float32),
            [pltpu.SemaphoreType.DMA for _ in range(4)],
        ],
    )
    def k(msg_hbm, dst_hbm, zero_hbm, out_hbm, dstv, mbufs, table, sems):
        c = lax.axis_index("c")
        s = lax.axis_index("s")
        nch = _EC // _MCH

        def chunk_cp(t, buf, sem):
            return pltpu.async_copy(
                msg_hbm.at[
                    pl.ds(s * _CS, _CS), pl.ds(c * _EC + t * _MCH, _MCH)
                ],
                buf,
                sem,
            )

        zc = pltpu.async_copy(zero_hbm, table, sems[2])
        dc = pltpu.async_copy(dst_hbm.at[c], dstv, sems[3])
        cps = [chunk_cp(0, mbufs[0], sems[0])]
        dc.wait()
        zc.wait()
        for t in range(nch):
            if t + 1 < nch:
                cps.append(chunk_cp(t + 1, mbufs[(t + 1) % 2], sems[(t + 1) % 2]))
            cps[t].wait()
            mbuf = mbufs[t % 2]

            def grp(i, _):
                d16 = dstv[pl.ds(t * _MCH + i * 16, 16)]
                for r in range(16):
                    vals = mbuf[r, pl.ds(i * 16, 16)]
                    rr = jnp.full((16,), r, jnp.int32)
                    plsc.addupdate_scatter(table, [rr, d16], vals)
                return _

            lax.fori_loop(0, _MCH // 16, grp, jnp.int32(0))
        pltpu.sync_copy(table, out_hbm.at[pl.ds(c * 256 + s * _CS, _CS)])

    return k(msgt, dst2, zrows)


def _msg_call(eap, hsrc, w1p, b1r, w2r, smat, tmat, cinp, cout, ones_cols):
    nk = 32
    wtot = 256

    def body(ea_ref, hs_ref, w1_ref, b1_ref, w2_ref, s_ref, t_ref, out_ref):
        eh = jnp.maximum(
            jnp.dot(ea_ref[...], w1_ref[...], preferred_element_type=jnp.float32)
            + b1_ref[...],
            0.0,
        )
        hs = hs_ref[...][:, :cinp]
        hsb = hs.astype(jnp.bfloat16)
        ehb = jnp.dot(
            eh.astype(jnp.bfloat16), s_ref[...],
            preferred_element_type=jnp.float32,
        )
        hst = jnp.dot(hsb, t_ref[...], preferred_element_type=jnp.float32)
        q = jnp.concatenate([(ehb * hst).astype(jnp.bfloat16), hsb], axis=1)
        msg = jnp.dot(q, w2_ref[...], preferred_element_type=jnp.float32)
        pieces = [msg]
        if ones_cols:
            pieces.append(jnp.ones((msg.shape[0], ones_cols), jnp.float32))
        pad = wtot - cout - ones_cols
        if pad:
            pieces.append(jnp.zeros((msg.shape[0], pad), jnp.float32))
        full = jnp.concatenate(pieces, axis=1) if len(pieces) > 1 else msg
        out_ref[...] = full.T

    return pl.pallas_call(
        body,
        grid=(_EP // _EB,),
        in_specs=[
            pl.BlockSpec((_EB, 8), lambda i: (i, 0)),
            pl.BlockSpec((_EB, 128), lambda i: (i, 0)),
            pl.BlockSpec((8, 32), lambda i: (0, 0)),
            pl.BlockSpec((1, 32), lambda i: (0, 0)),
            pl.BlockSpec(((nk + 1) * cinp, cout), lambda i: (0, 0)),
            pl.BlockSpec((nk, nk * cinp), lambda i: (0, 0)),
            pl.BlockSpec((cinp, nk * cinp), lambda i: (0, 0)),
        ],
        out_specs=pl.BlockSpec((wtot, _EB), lambda i: (0, i)),
        out_shape=jax.ShapeDtypeStruct((wtot, _EP), jnp.float32),
    )(eap, hsrc, w1p, b1r, w2r, smat, tmat)


def _node_call(h, rootp, parts, inv_or_cnt, biasr, gammar, betar, cinp, cout, first):
    wout = max(cout, 128)
    nblk = _NP // _NB

    def body(h_ref, root_ref, p0_ref, p1_ref, cv_ref, bias_ref,
             g_ref, beta_ref, out_ref, inv_ref):
        p0t = p0_ref[...]
        p1t = p1_ref[...]
        psum = (p0t[:cout, :] + p1t[:cout, :]).T
        if first:
            cntt = p0t[cout : cout + 16, :] + p1t[cout : cout + 16, :]
            cnt = cntt.T
            inv = 1.0 / jnp.maximum(cnt[:, :1], 1.0)
            inv_ref[...] = jnp.broadcast_to(inv, (_NB, 16))
        else:
            inv = cv_ref[...][:, :1]
        agg = psum * inv
        y = (
            jnp.dot(h_ref[...], root_ref[...], preferred_element_type=jnp.float32)
            + agg
            + bias_ref[...]
        )
        hv = jnp.maximum(y * g_ref[...] + beta_ref[...], 0.0)
        if wout > cout:
            hv = jnp.concatenate(
                [hv, jnp.zeros((_NB, wout - cout), jnp.float32)], axis=1
            )
        out_ref[...] = hv

    in_specs = [
        pl.BlockSpec((_NB, 128), lambda i: (i, 0)),
        pl.BlockSpec((128, cout), lambda i: (0, 0)),
        pl.BlockSpec((256, _NB), lambda i: (0, i)),
        pl.BlockSpec((256, _NB), lambda i: (1, i)),
        pl.BlockSpec((_NB, 16), lambda i: (i, 0)),
        pl.BlockSpec((1, cout), lambda i: (0, 0)),
        pl.BlockSpec((1, cout), lambda i: (0, 0)),
        pl.BlockSpec((1, cout), lambda i: (0, 0)),
    ]
    inv_in = jnp.zeros((_NP, 16), jnp.float32) if first else inv_or_cnt
    out = pl.pallas_call(
        body,
        grid=(nblk,),
        in_specs=in_specs,
        out_specs=[
            pl.BlockSpec((_NB, wout), lambda i: (i, 0)),
            pl.BlockSpec((_NB, 16), lambda i: (i, 0)),
        ],
        out_shape=[
            jax.ShapeDtypeStruct((_NP, wout), jnp.float32),
            jax.ShapeDtypeStruct((_NP, 16), jnp.float32),
        ],
    )(h, rootp, parts, parts, inv_in, biasr, gammar, betar)
    return out


def _node_pool_call(h, rootp, parts, inv, biasr, gammar, betar, bs3, wpp, bpp):
    nblk = _NP // _NB
    cout = 256

    def body(h_ref, root_ref, p0_ref, p1_ref, cv_ref, bias_ref, g_ref,
             beta_ref, bs_ref, wp_ref, bp_ref, out_ref, acc, pcnt):
        i = pl.program_id(0)

        @pl.when(i == 0)
        def _init():
            acc[...] = jnp.zeros_like(acc)
            pcnt[...] = jnp.zeros_like(pcnt)

        psum = (p0_ref[...] + p1_ref[...]).T
        invc = cv_ref[...][:, :1]
        y = (
            jnp.dot(h_ref[...], root_ref[...], preferred_element_type=jnp.float32)
            + psum * invc
            + bias_ref[...]
        )
        h3 = jnp.maximum(y * g_ref[...] + beta_ref[...], 0.0)
        seg = lax.broadcasted_iota(jnp.int32, (_NG, _NB), 0)
        bs = bs_ref[0]
        oh = (seg == bs).astype(jnp.float32)
        acc[...] += jnp.dot(oh, h3, preferred_element_type=jnp.float32)
        pcnt[...] += jnp.broadcast_to(
            jnp.sum(oh, axis=1, keepdims=True), (_NG, 128)
        )

        @pl.when(i == nblk - 1)
        def _fin():
            pooled = acc[...] * (1.0 / jnp.maximum(pcnt[...][:, :1], 1.0))
            o = jnp.dot(
                pooled, wp_ref[...], preferred_element_type=jnp.float32
            ) + bp_ref[...]
            out_ref[...] = jnp.where(o > 0, o, 0.1 * o)

    return pl.pallas_call(
        body,
        grid=(nblk,),
        in_specs=[
            pl.BlockSpec((_NB, 128), lambda i: (i, 0)),
            pl.BlockSpec((128, cout), lambda i: (0, 0)),
            pl.BlockSpec((256, _NB), lambda i: (0, i)),
            pl.BlockSpec((256, _NB), lambda i: (1, i)),
            pl.BlockSpec((_NB, 16), lambda i: (i, 0)),
            pl.BlockSpec((1, cout), lambda i: (0, 0)),
            pl.BlockSpec((1, cout), lambda i: (0, 0)),
            pl.BlockSpec((1, cout), lambda i: (0, 0)),
            pl.BlockSpec((1, 1, _NB), lambda i: (i, 0, 0)),
            pl.BlockSpec((256, 128), lambda i: (0, 0)),
            pl.BlockSpec((1, 128), lambda i: (0, 0)),
        ],
        out_specs=pl.BlockSpec((_NG, 128), lambda i: (0, 0)),
        out_shape=jax.ShapeDtypeStruct((_NG, 128), jnp.float32),
        scratch_shapes=[
            pltpu.VMEM((_NG, 256), jnp.float32),
            pltpu.VMEM((_NG, 128), jnp.float32),
        ],
    )(h, rootp, parts, parts, inv, biasr, gammar, betar, bs3, wpp, bpp)


def _prep_layer(p, cin, cinp, cout):
    w1p = jnp.zeros((8, 32), jnp.float32).at[:3].set(p["W1"])
    b1r = p["b1"].reshape(1, 32)
    w2 = p["W2"].reshape(32, cin, cout)
    b2r = jnp.zeros((cinp, cout), jnp.float32).at[:cin].set(
        p["b2"].reshape(cin, cout)
    )
    w2r = jnp.concatenate(
        [
            jnp.zeros((32, cinp, cout), jnp.float32)
            .at[:, :cin, :]
            .set(w2)
            .reshape(32 * cinp, cout),
            b2r,
        ],
        axis=0,
    ).astype(jnp.bfloat16)
    kk = jnp.arange(32 * cinp)
    smat = (kk[None, :] // cinp == jnp.arange(32)[:, None]).astype(jnp.bfloat16)
    tmat = (kk[None, :] % cinp == jnp.arange(cinp)[:, None]).astype(jnp.bfloat16)
    rootp = jnp.zeros((128, cout), jnp.float32).at[:cin].set(p["root"])
    biasr = p["bias"].reshape(1, cout)
    gammar = (p["gamma"] / jnp.sqrt(1.0 + 1e-5)).reshape(1, cout)
    betar = p["beta"].reshape(1, cout)
    return w1p, b1r, w2r, smat, tmat, rootp, biasr, gammar, betar


def kernel(x, edge_index, edge_attr, batch_seg, params):
    f32 = jnp.float32
    src = edge_index[0]
    dst = edge_index[1]
    xp = jnp.zeros((_NP, 128), f32).at[:_N, :5].set(x)
    src3 = (
        jnp.zeros((_EP,), jnp.int32).at[:_E].set(src).reshape(_NW, _NCHUNK, _CH)
    )
    dst2 = jnp.full((_EP,), _N, jnp.int32).at[:_E].set(dst).reshape(_NC, _EC)
    eap = jnp.zeros((_EP, 8), f32).at[:_E, :3].set(edge_attr)
    bs3 = (
        jnp.full((_NP,), _NG + 8, jnp.int32)
        .at[:_N]
        .set(batch_seg)
        .reshape(_NP // _NB, 1, _NB)
    )
    zrows = jnp.zeros((_CS, _NP), f32)
    l1 = _prep_layer(params["layer1"], 5, 16, 64)
    l2 = _prep_layer(params["layer2"], 64, 64, 128)
    l3 = _prep_layer(params["layer3"], 128, 128, 256)
    wpp = jnp.zeros((256, 128), f32).at[:, :_NT].set(params["mlp_W"])
    bpp = jnp.zeros((1, 128), f32).at[0, :_NT].set(params["mlp_b"])

    w1p, b1r, w2r, smat, tmat, rootp, biasr, gammar, betar = l1
    hs = _gather_call(xp, src3)
    msg = _msg_call(eap, hs, w1p, b1r, w2r, smat, tmat, 16, 64, 16)
    parts = _scatter_call(msg, dst2, zrows)
    h, inv = _node_call(xp, rootp, parts, None, biasr, gammar, betar, 16, 64, True)

    w1p, b1r, w2r, smat, tmat, rootp, biasr, gammar, betar = l2
    hs = _gather_call(h, src3)
    msg = _msg_call(eap, hs, w1p, b1r, w2r, smat, tmat, 64, 128, 0)
    parts = _scatter_call(msg, dst2, zrows)
    h, _ = _node_call(h, rootp, parts, inv, biasr, gammar, betar, 64, 128, False)

    w1p, b1r, w2r, smat, tmat, rootp, biasr, gammar, betar = l3
    hs = _gather_call(h, src3)
    msg = _msg_call(eap, hs, w1p, b1r, w2r, smat, tmat, 128, 256, 0)
    parts = _scatter_call(msg, dst2, zrows)

    out = _node_pool_call(
        h, rootp, parts, inv, biasr, gammar, betar, bs3, wpp, bpp
    )
    return out[:, :_NT]


# speedup vs baseline: 3.6750x; 1.0349x over previous
import functools

import jax
import jax.numpy as jnp
from jax import lax
from jax.experimental import pallas as pl
from jax.experimental.pallas import tpu as pltpu
from jax.experimental.pallas import tpu_sc as plsc

_N = 5000
_E = 10000
_NG = 256
_NT = 100

_NC = 2
_NS = 16
_NW = _NC * _NS
_CH = 80
_NCHUNK = 4
_TILE_E = _CH * _NCHUNK
_EP = _NW * _TILE_E
_NP = _NS * _TILE_E
_EB = 1024
_NB = 256


def _sc_mesh():
    return plsc.VectorSubcoreMesh(core_axis_name="c", subcore_axis_name="s")


def _gather_call(h, src3):
    cinp = 128

    @functools.partial(
        pl.kernel,
        out_type=jax.ShapeDtypeStruct((_EP, cinp), jnp.float32),
        mesh=_sc_mesh(),
        scratch_types=[
            pltpu.VMEM((_NCHUNK, _CH), jnp.int32),
            [pltpu.VMEM((_CH, cinp), jnp.float32) for _ in range(_NCHUNK)],
            [pltpu.SemaphoreType.DMA for _ in range(_NCHUNK)],
            [pltpu.SemaphoreType.DMA for _ in range(_NCHUNK)],
        ],
    )
    def k(h_hbm, src_hbm, out_hbm, idx_v, rows, gsems, wsems):
        c = lax.axis_index("c")
        s = lax.axis_index("s")
        wid = s * _NC + c
        pltpu.sync_copy(src_hbm.at[wid], idx_v)
        gcps = [
            pltpu.async_copy(h_hbm.at[idx_v.at[j]], rows[j], gsems[j])
            for j in range(_NCHUNK)
        ]
        wcps = []
        for j in range(_NCHUNK):
            gcps[j].wait()
            wcps.append(
                pltpu.async_copy(
                    rows[j],
                    out_hbm.at[pl.ds(wid * _TILE_E + j * _CH, _CH)],
                    wsems[j],
                )
            )
        for w in wcps:
            w.wait()

    return k(h, src3)


_EC = _EP // _NC
_CS = 16
_MCH = 1024


def _scatter_call(msgt, dst2, zrows):
    @functools.partial(
        pl.kernel,
        out_type=jax.ShapeDtypeStruct((2 * 256, _NP), jnp.float32),
        mesh=_sc_mesh(),
        compiler_params=pltpu.CompilerParams(needs_layout_passes=False),
        scratch_types=[
            pltpu.VMEM((_EC,), jnp.int32),
            [pltpu.VMEM((_CS, _MCH), jnp.float32) for _ in range(2)],
            pltpu.VMEM((_CS, _NP), jnp.float32),
            [pltpu.SemaphoreType.DMA for _ in range(4)],
        ],
    )
    def k(msg_hbm, dst_hbm, zero_hbm, out_hbm, dstv, mbufs, table, sems):
        c = lax.axis_index("c")
        s = lax.axis_index("s")
        nch = _EC // _MCH

        def chunk_cp(t, buf, sem):
            return pltpu.async_copy(
                msg_hbm.at[
                    pl.ds(s * _CS, _CS), pl.ds(c * _EC + t * _MCH, _MCH)
                ],
                buf,
                sem,
            )

        zc = pltpu.async_copy(zero_hbm, table, sems[2])
        dc = pltpu.async_copy(dst_hbm.at[c], dstv, sems[3])
        cps = [chunk_cp(0, mbufs[0], sems[0])]
        dc.wait()
        zc.wait()
        for t in range(nch):
            if t + 1 < nch:
                cps.append(chunk_cp(t + 1, mbufs[(t + 1) % 2], sems[(t + 1) % 2]))
            cps[t].wait()
            mbuf = mbufs[t % 2]

            def grp(i, _):
                d16 = dstv[pl.ds(t * _MCH + i * 16, 16)]
                for r in range(16):
                    vals = mbuf[r, pl.ds(i * 16, 16)]
                    rr = jnp.full((16,), r, jnp.int32)
                    plsc.addupdate_scatter(table, [rr, d16], vals)
                return _

            lax.fori_loop(0, _MCH // 16, grp, jnp.int32(0))
        pltpu.sync_copy(table, out_hbm.at[pl.ds(c * 256 + s * _CS, _CS)])

    return k(msgt, dst2, zrows)


def _msg_call(eap, hsrc, w1p, b1r, w2r, smat, tmat, cinp, cout, ones_cols):
    nk = 32
    wtot = 256

    def body(ea_ref, hs_ref, w1_ref, b1_ref, w2_ref, s_ref, t_ref, out_ref):
        eh = jnp.maximum(
            jnp.dot(ea_ref[...], w1_ref[...], preferred_element_type=jnp.float32)
            + b1_ref[...],
            0.0,
        )
        hs = hs_ref[...][:, :cinp]
        hsb = hs.astype(jnp.bfloat16)
        ehb = jnp.dot(
            eh.astype(jnp.bfloat16), s_ref[...],
            preferred_element_type=jnp.float32,
        )
        hst = jnp.dot(hsb, t_ref[...], preferred_element_type=jnp.float32)
        q = jnp.concatenate([(ehb * hst).astype(jnp.bfloat16), hsb], axis=1)
        msg = jnp.dot(q, w2_ref[...], preferred_element_type=jnp.float32)
        pieces = [msg]
        if ones_cols:
            pieces.append(jnp.ones((msg.shape[0], ones_cols), jnp.float32))
        pad = wtot - cout - ones_cols
        if pad:
            pieces.append(jnp.zeros((msg.shape[0], pad), jnp.float32))
        full = jnp.concatenate(pieces, axis=1) if len(pieces) > 1 else msg
        out_ref[...] = full.T

    return pl.pallas_call(
        body,
        grid=(_EP // _EB,),
        in_specs=[
            pl.BlockSpec((_EB, 8), lambda i: (i, 0)),
            pl.BlockSpec((_EB, 128), lambda i: (i, 0)),
            pl.BlockSpec((8, 32), lambda i: (0, 0)),
            pl.BlockSpec((1, 32), lambda i: (0, 0)),
            pl.BlockSpec(((nk + 1) * cinp, cout), lambda i: (0, 0)),
            pl.BlockSpec((nk, nk * cinp), lambda i: (0, 0)),
            pl.BlockSpec((cinp, nk * cinp), lambda i: (0, 0)),
        ],
        out_specs=pl.BlockSpec((wtot, _EB), lambda i: (0, i)),
        out_shape=jax.ShapeDtypeStruct((wtot, _EP), jnp.float32),
    )(eap, hsrc, w1p, b1r, w2r, smat, tmat)


def _node_call(h, rootp, parts, inv_or_cnt, biasr, gammar, betar, cinp, cout, first):
    wout = max(cout, 128)
    nblk = _NP // _NB

    def body(h_ref, root_ref, p0_ref, p1_ref, cv_ref, bias_ref,
             g_ref, beta_ref, out_ref, inv_ref):
        p0t = p0_ref[...]
        p1t = p1_ref[...]
        psum = (p0t[:cout, :] + p1t[:cout, :]).T
        if first:
            cntt = p0t[cout : cout + 16, :] + p1t[cout : cout + 16, :]
            cnt = cntt.T
            inv = 1.0 / jnp.maximum(cnt[:, :1], 1.0)
            inv_ref[...] = jnp.broadcast_to(inv, (_NB, 16))
        else:
            inv = cv_ref[...][:, :1]
        agg = psum * inv
        y = (
            jnp.dot(h_ref[...], root_ref[...], preferred_element_type=jnp.float32)
            + agg
            + bias_ref[...]
        )
        hv = jnp.maximum(y * g_ref[...] + beta_ref[...], 0.0)
        if wout > cout:
            hv = jnp.concatenate(
                [hv, jnp.zeros((_NB, wout - cout), jnp.float32)], axis=1
            )
        out_ref[...] = hv

    in_specs = [
        pl.BlockSpec((_NB, 128), lambda i: (i, 0)),
        pl.BlockSpec((128, cout), lambda i: (0, 0)),
        pl.BlockSpec((256, _NB), lambda i: (0, i)),
        pl.BlockSpec((256, _NB), lambda i: (1, i)),
        pl.BlockSpec((_NB, 16), lambda i: (i, 0)),
        pl.BlockSpec((1, cout), lambda i: (0, 0)),
        pl.BlockSpec((1, cout), lambda i: (0, 0)),
        pl.BlockSpec((1, cout), lambda i: (0, 0)),
    ]
    inv_in = jnp.zeros((_NP, 16), jnp.float32) if first else inv_or_cnt
    out = pl.pallas_call(
        body,
        grid=(nblk,),
        in_specs=in_specs,
        out_specs=[
            pl.BlockSpec((_NB, wout), lambda i: (i, 0)),
            pl.BlockSpec((_NB, 16), lambda i: (i, 0)),
        ],
        out_shape=[
            jax.ShapeDtypeStruct((_NP, wout), jnp.float32),
            jax.ShapeDtypeStruct((_NP, 16), jnp.float32),
        ],
    )(h, rootp, parts, parts, inv_in, biasr, gammar, betar)
    return out


def _node_pool_call(h, rootp, parts, inv, biasr, gammar, betar, bs3, wpp, bpp):
    nblk = _NP // _NB
    cout = 256

    def body(h_ref, root_ref, p0_ref, p1_ref, cv_ref, bias_ref, g_ref,
             beta_ref, bs_ref, wp_ref, bp_ref, out_ref, acc, pcnt):
        i = pl.program_id(0)

        @pl.when(i == 0)
        def _init():
            acc[...] = jnp.zeros_like(acc)
            pcnt[...] = jnp.zeros_like(pcnt)

        psum = (p0_ref[...] + p1_ref[...]).T
        invc = cv_ref[...][:, :1]
        y = (
            jnp.dot(h_ref[...], root_ref[...], preferred_element_type=jnp.float32)
            + psum * invc
            + bias_ref[...]
        )
        h3 = jnp.maximum(y * g_ref[...] + beta_ref[...], 0.0)
        seg = lax.broadcasted_iota(jnp.int32, (_NG, _NB), 0)
        bs = bs_ref[0]
        oh = (seg == bs).astype(jnp.float32)
        acc[...] += jnp.dot(oh, h3, preferred_element_type=jnp.float32)
        pcnt[...] += jnp.broadcast_to(
            jnp.sum(oh, axis=1, keepdims=True), (_NG, 128)
        )

        @pl.when(i == nblk - 1)
        def _fin():
            pooled = acc[...] * (1.0 / jnp.maximum(pcnt[...][:, :1], 1.0))
            o = jnp.dot(
                pooled, wp_ref[...], preferred_element_type=jnp.float32
            ) + bp_ref[...]
            out_ref[...] = jnp.where(o > 0, o, 0.1 * o)

    return pl.pallas_call(
        body,
        grid=(nblk,),
        in_specs=[
            pl.BlockSpec((_NB, 128), lambda i: (i, 0)),
            pl.BlockSpec((128, cout), lambda i: (0, 0)),
            pl.BlockSpec((256, _NB), lambda i: (0, i)),
            pl.BlockSpec((256, _NB), lambda i: (1, i)),
            pl.BlockSpec((_NB, 16), lambda i: (i, 0)),
            pl.BlockSpec((1, cout), lambda i: (0, 0)),
            pl.BlockSpec((1, cout), lambda i: (0, 0)),
            pl.BlockSpec((1, cout), lambda i: (0, 0)),
            pl.BlockSpec((1, 1, _NB), lambda i: (i, 0, 0)),
            pl.BlockSpec((256, 128), lambda i: (0, 0)),
            pl.BlockSpec((1, 128), lambda i: (0, 0)),
        ],
        out_specs=pl.BlockSpec((_NG, 128), lambda i: (0, 0)),
        out_shape=jax.ShapeDtypeStruct((_NG, 128), jnp.float32),
        scratch_shapes=[
            pltpu.VMEM((_NG, 256), jnp.float32),
            pltpu.VMEM((_NG, 128), jnp.float32),
        ],
    )(h, rootp, parts, parts, inv, biasr, gammar, betar, bs3, wpp, bpp)


def _prep_layer(p, cin, cinp, cout):
    w1p = jnp.zeros((8, 32), jnp.float32).at[:3].set(p["W1"])
    b1r = p["b1"].reshape(1, 32)
    w2 = p["W2"].reshape(32, cin, cout)
    b2r = jnp.zeros((cinp, cout), jnp.float32).at[:cin].set(
        p["b2"].reshape(cin, cout)
    )
    w2r = jnp.concatenate(
        [
            jnp.zeros((32, cinp, cout), jnp.float32)
            .at[:, :cin, :]
            .set(w2)
            .reshape(32 * cinp, cout),
            b2r,
        ],
        axis=0,
    ).astype(jnp.bfloat16)
    kk = jnp.arange(32 * cinp)
    smat = (kk[None, :] // cinp == jnp.arange(32)[:, None]).astype(jnp.bfloat16)
    tmat = (kk[None, :] % cinp == jnp.arange(cinp)[:, None]).astype(jnp.bfloat16)
    rootp = jnp.zeros((128, cout), jnp.float32).at[:cin].set(p["root"])
    biasr = p["bias"].reshape(1, cout)
    gammar = (p["gamma"] / jnp.sqrt(1.0 + 1e-5)).reshape(1, cout)
    betar = p["beta"].reshape(1, cout)
    return w1p, b1r, w2r, smat, tmat, rootp, biasr, gammar, betar


def kernel(x, edge_index, edge_attr, batch_seg, params):
    f32 = jnp.float32
    src = edge_index[0]
    dst = edge_index[1]
    xp = jnp.zeros((_NP, 128), f32).at[:_N, :5].set(x)
    src3 = (
        jnp.zeros((_EP,), jnp.int32).at[:_E].set(src).reshape(_NW, _NCHUNK, _CH)
    )
    dst2 = jnp.full((_EP,), _N, jnp.int32).at[:_E].set(dst).reshape(_NC, _EC)
    eap = jnp.zeros((_EP, 8), f32).at[:_E, :3].set(edge_attr)
    bs3 = (
        jnp.full((_NP,), _NG + 8, jnp.int32)
        .at[:_N]
        .set(batch_seg)
        .reshape(_NP // _NB, 1, _NB)
    )
    zrows = jnp.zeros((_CS, _NP), f32)
    l1 = _prep_layer(params["layer1"], 5, 16, 64)
    l2 = _prep_layer(params["layer2"], 64, 64, 128)
    l3 = _prep_layer(params["layer3"], 128, 128, 256)
    wpp = jnp.zeros((256, 128), f32).at[:, :_NT].set(params["mlp_W"])
    bpp = jnp.zeros((1, 128), f32).at[0, :_NT].set(params["mlp_b"])

    w1p, b1r, w2r, smat, tmat, rootp, biasr, gammar, betar = l1
    hs = _gather_call(xp, src3)
    msg = _msg_call(eap, hs, w1p, b1r, w2r, smat, tmat, 16, 64, 16)
    parts = _scatter_call(msg, dst2, zrows)
    h, inv = _node_call(xp, rootp, parts, None, biasr, gammar, betar, 16, 64, True)

    w1p, b1r, w2r, smat, tmat, rootp, biasr, gammar, betar = l2
    hs = _gather_call(h, src3)
    msg = _msg_call(eap, hs, w1p, b1r, w2r, smat, tmat, 64, 128, 0)
    parts = _scatter_call(msg, dst2, zrows)
    h, _ = _node_call(h, rootp, parts, inv, biasr, gammar, betar, 64, 128, False)

    w1p, b1r, w2r, smat, tmat, rootp, biasr, gammar, betar = l3
    hs = _gather_call(h, src3)
    msg = _msg_call(eap, hs, w1p, b1r, w2r, smat, tmat, 128, 256, 0)
    parts = _scatter_call(msg, dst2, zrows)

    out = _node_pool_call(
        h, rootp, parts, inv, biasr, gammar, betar, bs3, wpp, bpp
    )
    return out[:, :_NT]


# speedup vs baseline: 3.7127x; 1.0103x over previous
import functools

import jax
import jax.numpy as jnp
from jax import lax
from jax.experimental import pallas as pl
from jax.experimental.pallas import tpu as pltpu
from jax.experimental.pallas import tpu_sc as plsc

_N = 5000
_E = 10000
_NG = 256
_NT = 100

_NC = 2
_NS = 16
_NW = _NC * _NS
_CH = 80
_NCHUNK = 4
_TILE_E = _CH * _NCHUNK
_EP = _NW * _TILE_E
_NP = _NS * _TILE_E
_EB = 2048
_NB = 256


def _sc_mesh():
    return plsc.VectorSubcoreMesh(core_axis_name="c", subcore_axis_name="s")


def _gather_call(h, src3):
    cinp = 128

    @functools.partial(
        pl.kernel,
        out_type=jax.ShapeDtypeStruct((_EP, cinp), jnp.float32),
        mesh=_sc_mesh(),
        scratch_types=[
            pltpu.VMEM((_NCHUNK, _CH), jnp.int32),
            [pltpu.VMEM((_CH, cinp), jnp.float32) for _ in range(_NCHUNK)],
            [pltpu.SemaphoreType.DMA for _ in range(_NCHUNK)],
            [pltpu.SemaphoreType.DMA for _ in range(_NCHUNK)],
        ],
    )
    def k(h_hbm, src_hbm, out_hbm, idx_v, rows, gsems, wsems):
        c = lax.axis_index("c")
        s = lax.axis_index("s")
        wid = s * _NC + c
        pltpu.sync_copy(src_hbm.at[wid], idx_v)
        gcps = [
            pltpu.async_copy(h_hbm.at[idx_v.at[j]], rows[j], gsems[j])
            for j in range(_NCHUNK)
        ]
        wcps = []
        for j in range(_NCHUNK):
            gcps[j].wait()
            wcps.append(
                pltpu.async_copy(
                    rows[j],
                    out_hbm.at[pl.ds(wid * _TILE_E + j * _CH, _CH)],
                    wsems[j],
                )
            )
        for w in wcps:
            w.wait()

    return k(h, src3)


_EC = _EP // _NC
_CS = 16
_MCH = 1024


def _scatter_call(msgt, dst2, zrows):
    @functools.partial(
        pl.kernel,
        out_type=jax.ShapeDtypeStruct((2 * 256, _NP), jnp.float32),
        mesh=_sc_mesh(),
        compiler_params=pltpu.CompilerParams(needs_layout_passes=False),
        scratch_types=[
            pltpu.VMEM((_EC,), jnp.int32),
            [pltpu.VMEM((_CS, _MCH), jnp.float32) for _ in range(2)],
            pltpu.VMEM((_CS, _NP), jnp.float32),
            [pltpu.SemaphoreType.DMA for _ in range(4)],
        ],
    )
    def k(msg_hbm, dst_hbm, zero_hbm, out_hbm, dstv, mbufs, table, sems):
        c = lax.axis_index("c")
        s = lax.axis_index("s")
        nch = _EC // _MCH

        def chunk_cp(t, buf, sem):
            return pltpu.async_copy(
                msg_hbm.at[
                    pl.ds(s * _CS, _CS), pl.ds(c * _EC + t * _MCH, _MCH)
                ],
                buf,
                sem,
            )

        zc = pltpu.async_copy(zero_hbm, table, sems[2])
        dc = pltpu.async_copy(dst_hbm.at[c], dstv, sems[3])
        cps = [chunk_cp(0, mbufs[0], sems[0])]
        dc.wait()
        zc.wait()
        for t in range(nch):
            if t + 1 < nch:
                cps.append(chunk_cp(t + 1, mbufs[(t + 1) % 2], sems[(t + 1) % 2]))
            cps[t].wait()
            mbuf = mbufs[t % 2]

            def grp(i, _):
                d16 = dstv[pl.ds(t * _MCH + i * 16, 16)]
                for r in range(16):
                    vals = mbuf[r, pl.ds(i * 16, 16)]
                    rr = jnp.full((16,), r, jnp.int32)
                    plsc.addupdate_scatter(table, [rr, d16], vals)
                return _

            lax.fori_loop(0, _MCH // 16, grp, jnp.int32(0))
        pltpu.sync_copy(table, out_hbm.at[pl.ds(c * 256 + s * _CS, _CS)])

    return k(msgt, dst2, zrows)


def _msg_call(eap, hsrc, w1p, b1r, w2r, smat, tmat, cinp, cout, ones_cols):
    nk = 32
    wtot = 256

    def body(ea_ref, hs_ref, w1_ref, b1_ref, w2_ref, s_ref, t_ref, out_ref):
        eh = jnp.maximum(
            jnp.dot(ea_ref[...], w1_ref[...], preferred_element_type=jnp.float32)
            + b1_ref[...],
            0.0,
        )
        hs = hs_ref[...][:, :cinp]
        hsb = hs.astype(jnp.bfloat16)
        ehb = jnp.dot(
            eh.astype(jnp.bfloat16), s_ref[...],
            preferred_element_type=jnp.float32,
        )
        hst = jnp.dot(hsb, t_ref[...], preferred_element_type=jnp.float32)
        q = jnp.concatenate([(ehb * hst).astype(jnp.bfloat16), hsb], axis=1)
        msg = jnp.dot(q, w2_ref[...], preferred_element_type=jnp.float32)
        pieces = [msg]
        if ones_cols:
            pieces.append(jnp.ones((msg.shape[0], ones_cols), jnp.float32))
        pad = wtot - cout - ones_cols
        if pad:
            pieces.append(jnp.zeros((msg.shape[0], pad), jnp.float32))
        full = jnp.concatenate(pieces, axis=1) if len(pieces) > 1 else msg
        out_ref[...] = full.T

    return pl.pallas_call(
        body,
        grid=(_EP // _EB,),
        in_specs=[
            pl.BlockSpec((_EB, 8), lambda i: (i, 0)),
            pl.BlockSpec((_EB, 128), lambda i: (i, 0)),
            pl.BlockSpec((8, 32), lambda i: (0, 0)),
            pl.BlockSpec((1, 32), lambda i: (0, 0)),
            pl.BlockSpec(((nk + 1) * cinp, cout), lambda i: (0, 0)),
            pl.BlockSpec((nk, nk * cinp), lambda i: (0, 0)),
            pl.BlockSpec((cinp, nk * cinp), lambda i: (0, 0)),
        ],
        out_specs=pl.BlockSpec((wtot, _EB), lambda i: (0, i)),
        out_shape=jax.ShapeDtypeStruct((wtot, _EP), jnp.float32),
    )(eap, hsrc, w1p, b1r, w2r, smat, tmat)


def _node_call(h, rootp, parts, inv_or_cnt, biasr, gammar, betar, cinp, cout, first):
    wout = max(cout, 128)
    nblk = _NP // _NB

    def body(h_ref, root_ref, p0_ref, p1_ref, cv_ref, bias_ref,
             g_ref, beta_ref, out_ref, inv_ref):
        p0t = p0_ref[...]
        p1t = p1_ref[...]
        psum = (p0t[:cout, :] + p1t[:cout, :]).T
        if first:
            cntt = p0t[cout : cout + 16, :] + p1t[cout : cout + 16, :]
            cnt = cntt.T
            inv = 1.0 / jnp.maximum(cnt[:, :1], 1.0)
            inv_ref[...] = jnp.broadcast_to(inv, (_NB, 16))
        else:
            inv = cv_ref[...][:, :1]
        agg = psum * inv
        y = (
            jnp.dot(h_ref[...], root_ref[...], preferred_element_type=jnp.float32)
            + agg
            + bias_ref[...]
        )
        hv = jnp.maximum(y * g_ref[...] + beta_ref[...], 0.0)
        if wout > cout:
            hv = jnp.concatenate(
                [hv, jnp.zeros((_NB, wout - cout), jnp.float32)], axis=1
            )
        out_ref[...] = hv

    in_specs = [
        pl.BlockSpec((_NB, 128), lambda i: (i, 0)),
        pl.BlockSpec((128, cout), lambda i: (0, 0)),
        pl.BlockSpec((256, _NB), lambda i: (0, i)),
        pl.BlockSpec((256, _NB), lambda i: (1, i)),
        pl.BlockSpec((_NB, 16), lambda i: (i, 0)),
        pl.BlockSpec((1, cout), lambda i: (0, 0)),
        pl.BlockSpec((1, cout), lambda i: (0, 0)),
        pl.BlockSpec((1, cout), lambda i: (0, 0)),
    ]
    inv_in = jnp.zeros((_NP, 16), jnp.float32) if first else inv_or_cnt
    out = pl.pallas_call(
        body,
        grid=(nblk,),
        in_specs=in_specs,
        out_specs=[
            pl.BlockSpec((_NB, wout), lambda i: (i, 0)),
            pl.BlockSpec((_NB, 16), lambda i: (i, 0)),
        ],
        out_shape=[
            jax.ShapeDtypeStruct((_NP, wout), jnp.float32),
            jax.ShapeDtypeStruct((_NP, 16), jnp.float32),
        ],
    )(h, rootp, parts, parts, inv_in, biasr, gammar, betar)
    return out


def _node_pool_call(h, rootp, parts, inv, biasr, gammar, betar, bs3, wpp, bpp):
    nblk = _NP // _NB
    cout = 256

    def body(h_ref, root_ref, p0_ref, p1_ref, cv_ref, bias_ref, g_ref,
             beta_ref, bs_ref, wp_ref, bp_ref, out_ref, acc, pcnt):
        i = pl.program_id(0)

        @pl.when(i == 0)
        def _init():
            acc[...] = jnp.zeros_like(acc)
            pcnt[...] = jnp.zeros_like(pcnt)

        psum = (p0_ref[...] + p1_ref[...]).T
        invc = cv_ref[...][:, :1]
        y = (
            jnp.dot(h_ref[...], root_ref[...], preferred_element_type=jnp.float32)
            + psum * invc
            + bias_ref[...]
        )
        h3 = jnp.maximum(y * g_ref[...] + beta_ref[...], 0.0)
        seg = lax.broadcasted_iota(jnp.int32, (_NG, _NB), 0)
        bs = bs_ref[0]
        oh = (seg == bs).astype(jnp.float32)
        acc[...] += jnp.dot(oh, h3, preferred_element_type=jnp.float32)
        pcnt[...] += jnp.broadcast_to(
            jnp.sum(oh, axis=1, keepdims=True), (_NG, 128)
        )

        @pl.when(i == nblk - 1)
        def _fin():
            pooled = acc[...] * (1.0 / jnp.maximum(pcnt[...][:, :1], 1.0))
            o = jnp.dot(
                pooled, wp_ref[...], preferred_element_type=jnp.float32
            ) + bp_ref[...]
            out_ref[...] = jnp.where(o > 0, o, 0.1 * o)

    return pl.pallas_call(
        body,
        grid=(nblk,),
        in_specs=[
            pl.BlockSpec((_NB, 128), lambda i: (i, 0)),
            pl.BlockSpec((128, cout), lambda i: (0, 0)),
            pl.BlockSpec((256, _NB), lambda i: (0, i)),
            pl.BlockSpec((256, _NB), lambda i: (1, i)),
            pl.BlockSpec((_NB, 16), lambda i: (i, 0)),
            pl.BlockSpec((1, cout), lambda i: (0, 0)),
            pl.BlockSpec((1, cout), lambda i: (0, 0)),
            pl.BlockSpec((1, cout), lambda i: (0, 0)),
            pl.BlockSpec((1, 1, _NB), lambda i: (i, 0, 0)),
            pl.BlockSpec((256, 128), lambda i: (0, 0)),
            pl.BlockSpec((1, 128), lambda i: (0, 0)),
        ],
        out_specs=pl.BlockSpec((_NG, 128), lambda i: (0, 0)),
        out_shape=jax.ShapeDtypeStruct((_NG, 128), jnp.float32),
        scratch_shapes=[
            pltpu.VMEM((_NG, 256), jnp.float32),
            pltpu.VMEM((_NG, 128), jnp.float32),
        ],
    )(h, rootp, parts, parts, inv, biasr, gammar, betar, bs3, wpp, bpp)


def _prep_layer(p, cin, cinp, cout):
    w1p = jnp.zeros((8, 32), jnp.float32).at[:3].set(p["W1"])
    b1r = p["b1"].reshape(1, 32)
    w2 = p["W2"].reshape(32, cin, cout)
    b2r = jnp.zeros((cinp, cout), jnp.float32).at[:cin].set(
        p["b2"].reshape(cin, cout)
    )
    w2r = jnp.concatenate(
        [
            jnp.zeros((32, cinp, cout), jnp.float32)
            .at[:, :cin, :]
            .set(w2)
            .reshape(32 * cinp, cout),
            b2r,
        ],
        axis=0,
    ).astype(jnp.bfloat16)
    kk = jnp.arange(32 * cinp)
    smat = (kk[None, :] // cinp == jnp.arange(32)[:, None]).astype(jnp.bfloat16)
    tmat = (kk[None, :] % cinp == jnp.arange(cinp)[:, None]).astype(jnp.bfloat16)
    rootp = jnp.zeros((128, cout), jnp.float32).at[:cin].set(p["root"])
    biasr = p["bias"].reshape(1, cout)
    gammar = (p["gamma"] / jnp.sqrt(1.0 + 1e-5)).reshape(1, cout)
    betar = p["beta"].reshape(1, cout)
    return w1p, b1r, w2r, smat, tmat, rootp, biasr, gammar, betar


def kernel(x, edge_index, edge_attr, batch_seg, params):
    f32 = jnp.float32
    src = edge_index[0]
    dst = edge_index[1]
    xp = jnp.zeros((_NP, 128), f32).at[:_N, :5].set(x)
    src3 = (
        jnp.zeros((_EP,), jnp.int32).at[:_E].set(src).reshape(_NW, _NCHUNK, _CH)
    )
    dst2 = jnp.full((_EP,), _N, jnp.int32).at[:_E].set(dst).reshape(_NC, _EC)
    eap = jnp.zeros((_EP, 8), f32).at[:_E, :3].set(edge_attr)
    bs3 = (
        jnp.full((_NP,), _NG + 8, jnp.int32)
        .at[:_N]
        .set(batch_seg)
        .reshape(_NP // _NB, 1, _NB)
    )
    zrows = jnp.zeros((_CS, _NP), f32)
    l1 = _prep_layer(params["layer1"], 5, 16, 64)
    l2 = _prep_layer(params["layer2"], 64, 64, 128)
    l3 = _prep_layer(params["layer3"], 128, 128, 256)
    wpp = jnp.zeros((256, 128), f32).at[:, :_NT].set(params["mlp_W"])
    bpp = jnp.zeros((1, 128), f32).at[0, :_NT].set(params["mlp_b"])

    w1p, b1r, w2r, smat, tmat, rootp, biasr, gammar, betar = l1
    hs = _gather_call(xp, src3)
    msg = _msg_call(eap, hs, w1p, b1r, w2r, smat, tmat, 16, 64, 16)
    parts = _scatter_call(msg, dst2, zrows)
    h, inv = _node_call(xp, rootp, parts, None, biasr, gammar, betar, 16, 64, True)

    w1p, b1r, w2r, smat, tmat, rootp, biasr, gammar, betar = l2
    hs = _gather_call(h, src3)
    msg = _msg_call(eap, hs, w1p, b1r, w2r, smat, tmat, 64, 128, 0)
    parts = _scatter_call(msg, dst2, zrows)
    h, _ = _node_call(h, rootp, parts, inv, biasr, gammar, betar, 64, 128, False)

    w1p, b1r, w2r, smat, tmat, rootp, biasr, gammar, betar = l3
    hs = _gather_call(h, src3)
    msg = _msg_call(eap, hs, w1p, b1r, w2r, smat, tmat, 128, 256, 0)
    parts = _scatter_call(msg, dst2, zrows)

    out = _node_pool_call(
        h, rootp, parts, inv, biasr, gammar, betar, bs3, wpp, bpp
    )
    return out[:, :_NT]


# speedup vs baseline: 3.7129x; 1.0001x over previous
import functools

import jax
import jax.numpy as jnp
from jax import lax
from jax.experimental import pallas as pl
from jax.experimental.pallas import tpu as pltpu
from jax.experimental.pallas import tpu_sc as plsc

_N = 5000
_E = 10000
_NG = 256
_NT = 100

_NC = 2
_NS = 16
_NW = _NC * _NS
_CH = 80
_NCHUNK = 4
_TILE_E = _CH * _NCHUNK
_EP = _NW * _TILE_E
_NP = _NS * _TILE_E
_EB = 2048
_NB = 256


def _sc_mesh():
    return plsc.VectorSubcoreMesh(core_axis_name="c", subcore_axis_name="s")


def _gather_call(h, src3):
    cinp = 128

    @functools.partial(
        pl.kernel,
        out_type=jax.ShapeDtypeStruct((_EP, cinp), jnp.float32),
        mesh=_sc_mesh(),
        scratch_types=[
            pltpu.VMEM((_NCHUNK, _CH), jnp.int32),
            [pltpu.VMEM((_CH, cinp), jnp.float32) for _ in range(_NCHUNK)],
            [pltpu.SemaphoreType.DMA for _ in range(_NCHUNK)],
            [pltpu.SemaphoreType.DMA for _ in range(_NCHUNK)],
        ],
    )
    def k(h_hbm, src_hbm, out_hbm, idx_v, rows, gsems, wsems):
        c = lax.axis_index("c")
        s = lax.axis_index("s")
        wid = s * _NC + c
        pltpu.sync_copy(src_hbm.at[wid], idx_v)
        gcps = [
            pltpu.async_copy(h_hbm.at[idx_v.at[j]], rows[j], gsems[j])
            for j in range(_NCHUNK)
        ]
        wcps = []
        for j in range(_NCHUNK):
            gcps[j].wait()
            wcps.append(
                pltpu.async_copy(
                    rows[j],
                    out_hbm.at[pl.ds(wid * _TILE_E + j * _CH, _CH)],
                    wsems[j],
                )
            )
        for w in wcps:
            w.wait()

    return k(h, src3)


_EC = _EP // _NC
_CS = 16
_MCH = 1024


def _scatter_call(msgt, dst2, zrows):
    @functools.partial(
        pl.kernel,
        out_type=jax.ShapeDtypeStruct((2 * 256, _NP), jnp.float32),
        mesh=_sc_mesh(),
        compiler_params=pltpu.CompilerParams(needs_layout_passes=False),
        scratch_types=[
            pltpu.VMEM((_EC,), jnp.int32),
            [pltpu.VMEM((_CS, _MCH), jnp.float32) for _ in range(2)],
            [pltpu.VMEM((_CS // 2, _NP), jnp.float32) for _ in range(2)],
            [pltpu.SemaphoreType.DMA for _ in range(5)],
        ],
    )
    def k(msg_hbm, dst_hbm, zero_hbm, out_hbm, dstv, mbufs, tables, sems):
        c = lax.axis_index("c")
        s = lax.axis_index("s")
        nch = _EC // _MCH

        def chunk_cp(t, buf, sem):
            return pltpu.async_copy(
                msg_hbm.at[
                    pl.ds(s * _CS, _CS), pl.ds(c * _EC + t * _MCH, _MCH)
                ],
                buf,
                sem,
            )

        zc0 = pltpu.async_copy(zero_hbm.at[pl.ds(0, 8)], tables[0], sems[2])
        zc1 = pltpu.async_copy(zero_hbm.at[pl.ds(8, 8)], tables[1], sems[3])
        dc = pltpu.async_copy(dst_hbm.at[c], dstv, sems[4])
        cps = [chunk_cp(0, mbufs[0], sems[0])]
        dc.wait()
        zc0.wait()
        zc1.wait()
        for t in range(nch):
            if t + 1 < nch:
                cps.append(chunk_cp(t + 1, mbufs[(t + 1) % 2], sems[(t + 1) % 2]))
            cps[t].wait()
            mbuf = mbufs[t % 2]

            def grp(i, _):
                d16 = dstv[pl.ds(t * _MCH + i * 16, 16)]
                for r in range(8):
                    rr = jnp.full((16,), r, jnp.int32)
                    v0 = mbuf[r, pl.ds(i * 16, 16)]
                    plsc.addupdate_scatter(tables[0], [rr, d16], v0)
                    v1 = mbuf[r + 8, pl.ds(i * 16, 16)]
                    plsc.addupdate_scatter(tables[1], [rr, d16], v1)
                return _

            lax.fori_loop(0, _MCH // 16, grp, jnp.int32(0))
        pltpu.sync_copy(
            tables[0], out_hbm.at[pl.ds(c * 256 + s * _CS, _CS // 2)]
        )
        pltpu.sync_copy(
            tables[1], out_hbm.at[pl.ds(c * 256 + s * _CS + 8, _CS // 2)]
        )

    return k(msgt, dst2, zrows)


def _msg_call(eap, hsrc, w1p, b1r, w2r, smat, tmat, cinp, cout, ones_cols):
    nk = 32
    wtot = 256

    def body(ea_ref, hs_ref, w1_ref, b1_ref, w2_ref, s_ref, t_ref, out_ref):
        eh = jnp.maximum(
            jnp.dot(ea_ref[...], w1_ref[...], preferred_element_type=jnp.float32)
            + b1_ref[...],
            0.0,
        )
        hs = hs_ref[...][:, :cinp]
        hsb = hs.astype(jnp.bfloat16)
        ehb = jnp.dot(
            eh.astype(jnp.bfloat16), s_ref[...],
            preferred_element_type=jnp.float32,
        )
        hst = jnp.dot(hsb, t_ref[...], preferred_element_type=jnp.float32)
        q = jnp.concatenate([(ehb * hst).astype(jnp.bfloat16), hsb], axis=1)
        msg = jnp.dot(q, w2_ref[...], preferred_element_type=jnp.float32)
        pieces = [msg]
        if ones_cols:
            pieces.append(jnp.ones((msg.shape[0], ones_cols), jnp.float32))
        pad = wtot - cout - ones_cols
        if pad:
            pieces.append(jnp.zeros((msg.shape[0], pad), jnp.float32))
        full = jnp.concatenate(pieces, axis=1) if len(pieces) > 1 else msg
        out_ref[...] = full.T

    return pl.pallas_call(
        body,
        grid=(_EP // _EB,),
        in_specs=[
            pl.BlockSpec((_EB, 8), lambda i: (i, 0)),
            pl.BlockSpec((_EB, 128), lambda i: (i, 0)),
            pl.BlockSpec((8, 32), lambda i: (0, 0)),
            pl.BlockSpec((1, 32), lambda i: (0, 0)),
            pl.BlockSpec(((nk + 1) * cinp, cout), lambda i: (0, 0)),
            pl.BlockSpec((nk, nk * cinp), lambda i: (0, 0)),
            pl.BlockSpec((cinp, nk * cinp), lambda i: (0, 0)),
        ],
        out_specs=pl.BlockSpec((wtot, _EB), lambda i: (0, i)),
        out_shape=jax.ShapeDtypeStruct((wtot, _EP), jnp.float32),
    )(eap, hsrc, w1p, b1r, w2r, smat, tmat)


def _node_call(h, rootp, parts, inv_or_cnt, biasr, gammar, betar, cinp, cout, first):
    wout = max(cout, 128)
    nblk = _NP // _NB

    def body(h_ref, root_ref, p0_ref, p1_ref, cv_ref, bias_ref,
             g_ref, beta_ref, out_ref, inv_ref):
        p0t = p0_ref[...]
        p1t = p1_ref[...]
        psum = (p0t[:cout, :] + p1t[:cout, :]).T
        if first:
            cntt = p0t[cout : cout + 16, :] + p1t[cout : cout + 16, :]
            cnt = cntt.T
            inv = 1.0 / jnp.maximum(cnt[:, :1], 1.0)
            inv_ref[...] = jnp.broadcast_to(inv, (_NB, 16))
        else:
            inv = cv_ref[...][:, :1]
        agg = psum * inv
        y = (
            jnp.dot(h_ref[...], root_ref[...], preferred_element_type=jnp.float32)
            + agg
            + bias_ref[...]
        )
        hv = jnp.maximum(y * g_ref[...] + beta_ref[...], 0.0)
        if wout > cout:
            hv = jnp.concatenate(
                [hv, jnp.zeros((_NB, wout - cout), jnp.float32)], axis=1
            )
        out_ref[...] = hv

    in_specs = [
        pl.BlockSpec((_NB, 128), lambda i: (i, 0)),
        pl.BlockSpec((128, cout), lambda i: (0, 0)),
        pl.BlockSpec((256, _NB), lambda i: (0, i)),
        pl.BlockSpec((256, _NB), lambda i: (1, i)),
        pl.BlockSpec((_NB, 16), lambda i: (i, 0)),
        pl.BlockSpec((1, cout), lambda i: (0, 0)),
        pl.BlockSpec((1, cout), lambda i: (0, 0)),
        pl.BlockSpec((1, cout), lambda i: (0, 0)),
    ]
    inv_in = jnp.zeros((_NP, 16), jnp.float32) if first else inv_or_cnt
    out = pl.pallas_call(
        body,
        grid=(nblk,),
        in_specs=in_specs,
        out_specs=[
            pl.BlockSpec((_NB, wout), lambda i: (i, 0)),
            pl.BlockSpec((_NB, 16), lambda i: (i, 0)),
        ],
        out_shape=[
            jax.ShapeDtypeStruct((_NP, wout), jnp.float32),
            jax.ShapeDtypeStruct((_NP, 16), jnp.float32),
        ],
    )(h, rootp, parts, parts, inv_in, biasr, gammar, betar)
    return out


def _node_pool_call(h, rootp, parts, inv, biasr, gammar, betar, bs3, wpp, bpp):
    nblk = _NP // _NB
    cout = 256

    def body(h_ref, root_ref, p0_ref, p1_ref, cv_ref, bias_ref, g_ref,
             beta_ref, bs_ref, wp_ref, bp_ref, out_ref, acc, pcnt):
        i = pl.program_id(0)

        @pl.when(i == 0)
        def _init():
            acc[...] = jnp.zeros_like(acc)
            pcnt[...] = jnp.zeros_like(pcnt)

        psum = (p0_ref[...] + p1_ref[...]).T
        invc = cv_ref[...][:, :1]
        y = (
            jnp.dot(h_ref[...], root_ref[...], preferred_element_type=jnp.float32)
            + psum * invc
            + bias_ref[...]
        )
        h3 = jnp.maximum(y * g_ref[...] + beta_ref[...], 0.0)
        seg = lax.broadcasted_iota(jnp.int32, (_NG, _NB), 0)
        bs = bs_ref[0]
        oh = (seg == bs).astype(jnp.float32)
        acc[...] += jnp.dot(oh, h3, preferred_element_type=jnp.float32)
        pcnt[...] += jnp.broadcast_to(
            jnp.sum(oh, axis=1, keepdims=True), (_NG, 128)
        )

        @pl.when(i == nblk - 1)
        def _fin():
            pooled = acc[...] * (1.0 / jnp.maximum(pcnt[...][:, :1], 1.0))
            o = jnp.dot(
                pooled, wp_ref[...], preferred_element_type=jnp.float32
            ) + bp_ref[...]
            out_ref[...] = jnp.where(o > 0, o, 0.1 * o)

    return pl.pallas_call(
        body,
        grid=(nblk,),
        in_specs=[
            pl.BlockSpec((_NB, 128), lambda i: (i, 0)),
            pl.BlockSpec((128, cout), lambda i: (0, 0)),
            pl.BlockSpec((256, _NB), lambda i: (0, i)),
            pl.BlockSpec((256, _NB), lambda i: (1, i)),
            pl.BlockSpec((_NB, 16), lambda i: (i, 0)),
            pl.BlockSpec((1, cout), lambda i: (0, 0)),
            pl.BlockSpec((1, cout), lambda i: (0, 0)),
            pl.BlockSpec((1, cout), lambda i: (0, 0)),
            pl.BlockSpec((1, 1, _NB), lambda i: (i, 0, 0)),
            pl.BlockSpec((256, 128), lambda i: (0, 0)),
            pl.BlockSpec((1, 128), lambda i: (0, 0)),
        ],
        out_specs=pl.BlockSpec((_NG, 128), lambda i: (0, 0)),
        out_shape=jax.ShapeDtypeStruct((_NG, 128), jnp.float32),
        scratch_shapes=[
            pltpu.VMEM((_NG, 256), jnp.float32),
            pltpu.VMEM((_NG, 128), jnp.float32),
        ],
    )(h, rootp, parts, parts, inv, biasr, gammar, betar, bs3, wpp, bpp)


def _prep_layer(p, cin, cinp, cout):
    w1p = jnp.zeros((8, 32), jnp.float32).at[:3].set(p["W1"])
    b1r = p["b1"].reshape(1, 32)
    w2 = p["W2"].reshape(32, cin, cout)
    b2r = jnp.zeros((cinp, cout), jnp.float32).at[:cin].set(
        p["b2"].reshape(cin, cout)
    )
    w2r = jnp.concatenate(
        [
            jnp.zeros((32, cinp, cout), jnp.float32)
            .at[:, :cin, :]
            .set(w2)
            .reshape(32 * cinp, cout),
            b2r,
        ],
        axis=0,
    ).astype(jnp.bfloat16)
    kk = jnp.arange(32 * cinp)
    smat = (kk[None, :] // cinp == jnp.arange(32)[:, None]).astype(jnp.bfloat16)
    tmat = (kk[None, :] % cinp == jnp.arange(cinp)[:, None]).astype(jnp.bfloat16)
    rootp = jnp.zeros((128, cout), jnp.float32).at[:cin].set(p["root"])
    biasr = p["bias"].reshape(1, cout)
    gammar = (p["gamma"] / jnp.sqrt(1.0 + 1e-5)).reshape(1, cout)
    betar = p["beta"].reshape(1, cout)
    return w1p, b1r, w2r, smat, tmat, rootp, biasr, gammar, betar


def kernel(x, edge_index, edge_attr, batch_seg, params):
    f32 = jnp.float32
    src = edge_index[0]
    dst = edge_index[1]
    xp = jnp.zeros((_NP, 128), f32).at[:_N, :5].set(x)
    src3 = (
        jnp.zeros((_EP,), jnp.int32).at[:_E].set(src).reshape(_NW, _NCHUNK, _CH)
    )
    dst2 = jnp.full((_EP,), _N, jnp.int32).at[:_E].set(dst).reshape(_NC, _EC)
    eap = jnp.zeros((_EP, 8), f32).at[:_E, :3].set(edge_attr)
    bs3 = (
        jnp.full((_NP,), _NG + 8, jnp.int32)
        .at[:_N]
        .set(batch_seg)
        .reshape(_NP // _NB, 1, _NB)
    )
    zrows = jnp.zeros((_CS, _NP), f32)
    l1 = _prep_layer(params["layer1"], 5, 16, 64)
    l2 = _prep_layer(params["layer2"], 64, 64, 128)
    l3 = _prep_layer(params["layer3"], 128, 128, 256)
    wpp = jnp.zeros((256, 128), f32).at[:, :_NT].set(params["mlp_W"])
    bpp = jnp.zeros((1, 128), f32).at[0, :_NT].set(params["mlp_b"])

    w1p, b1r, w2r, smat, tmat, rootp, biasr, gammar, betar = l1
    hs = _gather_call(xp, src3)
    msg = _msg_call(eap, hs, w1p, b1r, w2r, smat, tmat, 16, 64, 16)
    parts = _scatter_call(msg, dst2, zrows)
    h, inv = _node_call(xp, rootp, parts, None, biasr, gammar, betar, 16, 64, True)

    w1p, b1r, w2r, smat, tmat, rootp, biasr, gammar, betar = l2
    hs = _gather_call(h, src3)
    msg = _msg_call(eap, hs, w1p, b1r, w2r, smat, tmat, 64, 128, 0)
    parts = _scatter_call(msg, dst2, zrows)
    h, _ = _node_call(h, rootp, parts, inv, biasr, gammar, betar, 64, 128, False)

    w1p, b1r, w2r, smat, tmat, rootp, biasr, gammar, betar = l3
    hs = _gather_call(h, src3)
    msg = _msg_call(eap, hs, w1p, b1r, w2r, smat, tmat, 128, 256, 0)
    parts = _scatter_call(msg, dst2, zrows)

    out = _node_pool_call(
        h, rootp, parts, inv, biasr, gammar, betar, bs3, wpp, bpp
    )
    return out[:, :_NT]


# speedup vs baseline: 3.8241x; 1.0300x over previous
import functools

import jax
import jax.numpy as jnp
from jax import lax
from jax.experimental import pallas as pl
from jax.experimental.pallas import tpu as pltpu
from jax.experimental.pallas import tpu_sc as plsc

_N = 5000
_E = 10000
_NG = 256
_NT = 100

_NC = 2
_NS = 16
_NW = _NC * _NS
_CH = 80
_NCHUNK = 4
_TILE_E = _CH * _NCHUNK
_EP = _NW * _TILE_E
_NP = _NS * _TILE_E
_EB = 2048
_NB = 256


def _sc_mesh():
    return plsc.VectorSubcoreMesh(core_axis_name="c", subcore_axis_name="s")


def _gather_call(h, src3):
    cinp = 128

    @functools.partial(
        pl.kernel,
        out_type=jax.ShapeDtypeStruct((_EP, cinp), jnp.float32),
        mesh=_sc_mesh(),
        scratch_types=[
            pltpu.VMEM((_NCHUNK, _CH), jnp.int32),
            [pltpu.VMEM((_CH, cinp), jnp.float32) for _ in range(_NCHUNK)],
            [pltpu.SemaphoreType.DMA for _ in range(_NCHUNK)],
            [pltpu.SemaphoreType.DMA for _ in range(_NCHUNK)],
        ],
    )
    def k(h_hbm, src_hbm, out_hbm, idx_v, rows, gsems, wsems):
        c = lax.axis_index("c")
        s = lax.axis_index("s")
        wid = s * _NC + c
        pltpu.sync_copy(src_hbm.at[wid], idx_v)
        gcps = [
            pltpu.async_copy(h_hbm.at[idx_v.at[j]], rows[j], gsems[j])
            for j in range(_NCHUNK)
        ]
        wcps = []
        for j in range(_NCHUNK):
            gcps[j].wait()
            wcps.append(
                pltpu.async_copy(
                    rows[j],
                    out_hbm.at[pl.ds(wid * _TILE_E + j * _CH, _CH)],
                    wsems[j],
                )
            )
        for w in wcps:
            w.wait()

    return k(h, src3)


_EC = _EP // _NC
_CS = 16
_MCH = 1024


def _scatter_call(msgt, dst2, zrows, wc):
    @functools.partial(
        pl.kernel,
        out_type=jax.ShapeDtypeStruct((2 * 256, _NP), jnp.float32),
        mesh=_sc_mesh(),
        compiler_params=pltpu.CompilerParams(needs_layout_passes=False),
        scratch_types=[
            pltpu.VMEM((_EC,), jnp.int32),
            [pltpu.VMEM((_CS, _MCH), jnp.float32) for _ in range(2)],
            [pltpu.VMEM((_CS // 2, _NP), jnp.float32) for _ in range(2)],
            [pltpu.SemaphoreType.DMA for _ in range(5)],
        ],
    )
    def k(msg_hbm, dst_hbm, zero_hbm, out_hbm, dstv, mbufs, tables, sems):
        c = lax.axis_index("c")
        s = lax.axis_index("s")
        nch = _EC // _MCH

        @pl.when(s * _CS < wc)
        def _active():
            _scatter_body(
                msg_hbm, dst_hbm, zero_hbm, out_hbm, dstv, mbufs, tables,
                sems, c, s, nch,
            )

    def _scatter_body(msg_hbm, dst_hbm, zero_hbm, out_hbm, dstv, mbufs,
                      tables, sems, c, s, nch):

        def chunk_cp(t, buf, sem):
            return pltpu.async_copy(
                msg_hbm.at[
                    pl.ds(s * _CS, _CS), pl.ds(c * _EC + t * _MCH, _MCH)
                ],
                buf,
                sem,
            )

        zc0 = pltpu.async_copy(zero_hbm.at[pl.ds(0, 8)], tables[0], sems[2])
        zc1 = pltpu.async_copy(zero_hbm.at[pl.ds(8, 8)], tables[1], sems[3])
        dc = pltpu.async_copy(dst_hbm.at[c], dstv, sems[4])
        cps = [chunk_cp(0, mbufs[0], sems[0])]
        dc.wait()
        zc0.wait()
        zc1.wait()
        for t in range(nch):
            if t + 1 < nch:
                cps.append(chunk_cp(t + 1, mbufs[(t + 1) % 2], sems[(t + 1) % 2]))
            cps[t].wait()
            mbuf = mbufs[t % 2]

            def grp(i, _):
                d16 = dstv[pl.ds(t * _MCH + i * 16, 16)]
                for r in range(8):
                    rr = jnp.full((16,), r, jnp.int32)
                    v0 = mbuf[r, pl.ds(i * 16, 16)]
                    plsc.addupdate_scatter(tables[0], [rr, d16], v0)
                    v1 = mbuf[r + 8, pl.ds(i * 16, 16)]
                    plsc.addupdate_scatter(tables[1], [rr, d16], v1)
                return _

            lax.fori_loop(0, _MCH // 16, grp, jnp.int32(0))
        pltpu.sync_copy(
            tables[0], out_hbm.at[pl.ds(c * 256 + s * _CS, _CS // 2)]
        )
        pltpu.sync_copy(
            tables[1], out_hbm.at[pl.ds(c * 256 + s * _CS + 8, _CS // 2)]
        )

    return k(msgt, dst2, zrows)


def _msg_call(eap, hsrc, w1p, b1r, w2r, smat, tmat, cinp, cout, ones_cols):
    nk = 32
    wtot = 256

    def body(ea_ref, hs_ref, w1_ref, b1_ref, w2_ref, s_ref, t_ref, out_ref):
        eh = jnp.maximum(
            jnp.dot(ea_ref[...], w1_ref[...], preferred_element_type=jnp.float32)
            + b1_ref[...],
            0.0,
        )
        hs = hs_ref[...][:, :cinp]
        hsb = hs.astype(jnp.bfloat16)
        ehb = jnp.dot(
            eh.astype(jnp.bfloat16), s_ref[...],
            preferred_element_type=jnp.float32,
        )
        hst = jnp.dot(hsb, t_ref[...], preferred_element_type=jnp.float32)
        q = jnp.concatenate([(ehb * hst).astype(jnp.bfloat16), hsb], axis=1)
        msg = jnp.dot(q, w2_ref[...], preferred_element_type=jnp.float32)
        pieces = [msg]
        if ones_cols:
            pieces.append(jnp.ones((msg.shape[0], ones_cols), jnp.float32))
        pad = wtot - cout - ones_cols
        if pad:
            pieces.append(jnp.zeros((msg.shape[0], pad), jnp.float32))
        full = jnp.concatenate(pieces, axis=1) if len(pieces) > 1 else msg
        out_ref[...] = full.T

    return pl.pallas_call(
        body,
        grid=(_EP // _EB,),
        in_specs=[
            pl.BlockSpec((_EB, 8), lambda i: (i, 0)),
            pl.BlockSpec((_EB, 128), lambda i: (i, 0)),
            pl.BlockSpec((8, 32), lambda i: (0, 0)),
            pl.BlockSpec((1, 32), lambda i: (0, 0)),
            pl.BlockSpec(((nk + 1) * cinp, cout), lambda i: (0, 0)),
            pl.BlockSpec((nk, nk * cinp), lambda i: (0, 0)),
            pl.BlockSpec((cinp, nk * cinp), lambda i: (0, 0)),
        ],
        out_specs=pl.BlockSpec((wtot, _EB), lambda i: (0, i)),
        out_shape=jax.ShapeDtypeStruct((wtot, _EP), jnp.float32),
    )(eap, hsrc, w1p, b1r, w2r, smat, tmat)


def _node_call(h, rootp, parts, inv_or_cnt, biasr, gammar, betar, cinp, cout, first):
    wout = max(cout, 128)
    nblk = _NP // _NB

    def body(h_ref, root_ref, p0_ref, p1_ref, cv_ref, bias_ref,
             g_ref, beta_ref, out_ref, inv_ref):
        p0t = p0_ref[...]
        p1t = p1_ref[...]
        psum = (p0t[:cout, :] + p1t[:cout, :]).T
        if first:
            cntt = p0t[cout : cout + 16, :] + p1t[cout : cout + 16, :]
            cnt = cntt.T
            inv = 1.0 / jnp.maximum(cnt[:, :1], 1.0)
            inv_ref[...] = jnp.broadcast_to(inv, (_NB, 16))
        else:
            inv = cv_ref[...][:, :1]
        agg = psum * inv
        y = (
            jnp.dot(h_ref[...], root_ref[...], preferred_element_type=jnp.float32)
            + agg
            + bias_ref[...]
        )
        hv = jnp.maximum(y * g_ref[...] + beta_ref[...], 0.0)
        if wout > cout:
            hv = jnp.concatenate(
                [hv, jnp.zeros((_NB, wout - cout), jnp.float32)], axis=1
            )
        out_ref[...] = hv

    in_specs = [
        pl.BlockSpec((_NB, 128), lambda i: (i, 0)),
        pl.BlockSpec((128, cout), lambda i: (0, 0)),
        pl.BlockSpec((256, _NB), lambda i: (0, i)),
        pl.BlockSpec((256, _NB), lambda i: (1, i)),
        pl.BlockSpec((_NB, 16), lambda i: (i, 0)),
        pl.BlockSpec((1, cout), lambda i: (0, 0)),
        pl.BlockSpec((1, cout), lambda i: (0, 0)),
        pl.BlockSpec((1, cout), lambda i: (0, 0)),
    ]
    inv_in = jnp.zeros((_NP, 16), jnp.float32) if first else inv_or_cnt
    out = pl.pallas_call(
        body,
        grid=(nblk,),
        in_specs=in_specs,
        out_specs=[
            pl.BlockSpec((_NB, wout), lambda i: (i, 0)),
            pl.BlockSpec((_NB, 16), lambda i: (i, 0)),
        ],
        out_shape=[
            jax.ShapeDtypeStruct((_NP, wout), jnp.float32),
            jax.ShapeDtypeStruct((_NP, 16), jnp.float32),
        ],
    )(h, rootp, parts, parts, inv_in, biasr, gammar, betar)
    return out


def _node_pool_call(h, rootp, parts, inv, biasr, gammar, betar, bs3, wpp, bpp):
    nblk = _NP // _NB
    cout = 256

    def body(h_ref, root_ref, p0_ref, p1_ref, cv_ref, bias_ref, g_ref,
             beta_ref, bs_ref, wp_ref, bp_ref, out_ref, acc, pcnt):
        i = pl.program_id(0)

        @pl.when(i == 0)
        def _init():
            acc[...] = jnp.zeros_like(acc)
            pcnt[...] = jnp.zeros_like(pcnt)

        psum = (p0_ref[...] + p1_ref[...]).T
        invc = cv_ref[...][:, :1]
        y = (
            jnp.dot(h_ref[...], root_ref[...], preferred_element_type=jnp.float32)
            + psum * invc
            + bias_ref[...]
        )
        h3 = jnp.maximum(y * g_ref[...] + beta_ref[...], 0.0)
        seg = lax.broadcasted_iota(jnp.int32, (_NG, _NB), 0)
        bs = bs_ref[0]
        oh = (seg == bs).astype(jnp.float32)
        acc[...] += jnp.dot(oh, h3, preferred_element_type=jnp.float32)
        pcnt[...] += jnp.broadcast_to(
            jnp.sum(oh, axis=1, keepdims=True), (_NG, 128)
        )

        @pl.when(i == nblk - 1)
        def _fin():
            pooled = acc[...] * (1.0 / jnp.maximum(pcnt[...][:, :1], 1.0))
            o = jnp.dot(
                pooled, wp_ref[...], preferred_element_type=jnp.float32
            ) + bp_ref[...]
            out_ref[...] = jnp.where(o > 0, o, 0.1 * o)

    return pl.pallas_call(
        body,
        grid=(nblk,),
        in_specs=[
            pl.BlockSpec((_NB, 128), lambda i: (i, 0)),
            pl.BlockSpec((128, cout), lambda i: (0, 0)),
            pl.BlockSpec((256, _NB), lambda i: (0, i)),
            pl.BlockSpec((256, _NB), lambda i: (1, i)),
            pl.BlockSpec((_NB, 16), lambda i: (i, 0)),
            pl.BlockSpec((1, cout), lambda i: (0, 0)),
            pl.BlockSpec((1, cout), lambda i: (0, 0)),
            pl.BlockSpec((1, cout), lambda i: (0, 0)),
            pl.BlockSpec((1, 1, _NB), lambda i: (i, 0, 0)),
            pl.BlockSpec((256, 128), lambda i: (0, 0)),
            pl.BlockSpec((1, 128), lambda i: (0, 0)),
        ],
        out_specs=pl.BlockSpec((_NG, 128), lambda i: (0, 0)),
        out_shape=jax.ShapeDtypeStruct((_NG, 128), jnp.float32),
        scratch_shapes=[
            pltpu.VMEM((_NG, 256), jnp.float32),
            pltpu.VMEM((_NG, 128), jnp.float32),
        ],
    )(h, rootp, parts, parts, inv, biasr, gammar, betar, bs3, wpp, bpp)


def _prep_layer(p, cin, cinp, cout):
    w1p = jnp.zeros((8, 32), jnp.float32).at[:3].set(p["W1"])
    b1r = p["b1"].reshape(1, 32)
    w2 = p["W2"].reshape(32, cin, cout)
    b2r = jnp.zeros((cinp, cout), jnp.float32).at[:cin].set(
        p["b2"].reshape(cin, cout)
    )
    w2r = jnp.concatenate(
        [
            jnp.zeros((32, cinp, cout), jnp.float32)
            .at[:, :cin, :]
            .set(w2)
            .reshape(32 * cinp, cout),
            b2r,
        ],
        axis=0,
    ).astype(jnp.bfloat16)
    kk = jnp.arange(32 * cinp)
    smat = (kk[None, :] // cinp == jnp.arange(32)[:, None]).astype(jnp.bfloat16)
    tmat = (kk[None, :] % cinp == jnp.arange(cinp)[:, None]).astype(jnp.bfloat16)
    rootp = jnp.zeros((128, cout), jnp.float32).at[:cin].set(p["root"])
    biasr = p["bias"].reshape(1, cout)
    gammar = (p["gamma"] / jnp.sqrt(1.0 + 1e-5)).reshape(1, cout)
    betar = p["beta"].reshape(1, cout)
    return w1p, b1r, w2r, smat, tmat, rootp, biasr, gammar, betar


def kernel(x, edge_index, edge_attr, batch_seg, params):
    f32 = jnp.float32
    src = edge_index[0]
    dst = edge_index[1]
    xp = jnp.zeros((_NP, 128), f32).at[:_N, :5].set(x)
    src3 = (
        jnp.zeros((_EP,), jnp.int32).at[:_E].set(src).reshape(_NW, _NCHUNK, _CH)
    )
    dst2 = jnp.full((_EP,), _N, jnp.int32).at[:_E].set(dst).reshape(_NC, _EC)
    eap = jnp.zeros((_EP, 8), f32).at[:_E, :3].set(edge_attr)
    bs3 = (
        jnp.full((_NP,), _NG + 8, jnp.int32)
        .at[:_N]
        .set(batch_seg)
        .reshape(_NP // _NB, 1, _NB)
    )
    zrows = jnp.zeros((_CS, _NP), f32)
    l1 = _prep_layer(params["layer1"], 5, 16, 64)
    l2 = _prep_layer(params["layer2"], 64, 64, 128)
    l3 = _prep_layer(params["layer3"], 128, 128, 256)
    wpp = jnp.zeros((256, 128), f32).at[:, :_NT].set(params["mlp_W"])
    bpp = jnp.zeros((1, 128), f32).at[0, :_NT].set(params["mlp_b"])

    w1p, b1r, w2r, smat, tmat, rootp, biasr, gammar, betar = l1
    hs = _gather_call(xp, src3)
    msg = _msg_call(eap, hs, w1p, b1r, w2r, smat, tmat, 16, 64, 16)
    parts = _scatter_call(msg, dst2, zrows, 80)
    h, inv = _node_call(xp, rootp, parts, None, biasr, gammar, betar, 16, 64, True)

    w1p, b1r, w2r, smat, tmat, rootp, biasr, gammar, betar = l2
    hs = _gather_call(h, src3)
    msg = _msg_call(eap, hs, w1p, b1r, w2r, smat, tmat, 64, 128, 0)
    parts = _scatter_call(msg, dst2, zrows, 128)
    h, _ = _node_call(h, rootp, parts, inv, biasr, gammar, betar, 64, 128, False)

    w1p, b1r, w2r, smat, tmat, rootp, biasr, gammar, betar = l3
    hs = _gather_call(h, src3)
    msg = _msg_call(eap, hs, w1p, b1r, w2r, smat, tmat, 128, 256, 0)
    parts = _scatter_call(msg, dst2, zrows, 256)

    out = _node_pool_call(
        h, rootp, parts, inv, biasr, gammar, betar, bs3, wpp, bpp
    )
    return out[:, :_NT]


# speedup vs baseline: 3.8435x; 1.0051x over previous
import functools

import jax
import jax.numpy as jnp
from jax import lax
from jax.experimental import pallas as pl
from jax.experimental.pallas import tpu as pltpu
from jax.experimental.pallas import tpu_sc as plsc

_N = 5000
_E = 10000
_NG = 256
_NT = 100

_NC = 2
_NS = 16
_NW = _NC * _NS
_CH = 80
_NCHUNK = 4
_TILE_E = _CH * _NCHUNK
_EP = _NW * _TILE_E
_NP = _NS * _TILE_E
_EB = 2048
_NB = 256


def _sc_mesh():
    return plsc.VectorSubcoreMesh(core_axis_name="c", subcore_axis_name="s")


def _gather_call(h, src3):
    cinp = 128

    @functools.partial(
        pl.kernel,
        out_type=jax.ShapeDtypeStruct((_EP, cinp), jnp.float32),
        mesh=_sc_mesh(),
        scratch_types=[
            pltpu.VMEM((_NCHUNK, _CH), jnp.int32),
            [pltpu.VMEM((_CH, cinp), jnp.float32) for _ in range(_NCHUNK)],
            [pltpu.SemaphoreType.DMA for _ in range(_NCHUNK)],
            [pltpu.SemaphoreType.DMA for _ in range(_NCHUNK)],
        ],
    )
    def k(h_hbm, src_hbm, out_hbm, idx_v, rows, gsems, wsems):
        c = lax.axis_index("c")
        s = lax.axis_index("s")
        wid = s * _NC + c
        pltpu.sync_copy(src_hbm.at[wid], idx_v)
        gcps = [
            pltpu.async_copy(h_hbm.at[idx_v.at[j]], rows[j], gsems[j])
            for j in range(_NCHUNK)
        ]
        wcps = []
        for j in range(_NCHUNK):
            gcps[j].wait()
            wcps.append(
                pltpu.async_copy(
                    rows[j],
                    out_hbm.at[pl.ds(wid * _TILE_E + j * _CH, _CH)],
                    wsems[j],
                )
            )
        for w in wcps:
            w.wait()

    return k(h, src3)


_EC = _EP // _NC
_CS = 16
_MCH = 1024


def _scatter_call(msgt, dst2, zrows, wc):
    @functools.partial(
        pl.kernel,
        out_type=jax.ShapeDtypeStruct((2 * wc, _NP), jnp.float32),
        mesh=_sc_mesh(),
        compiler_params=pltpu.CompilerParams(needs_layout_passes=False),
        scratch_types=[
            pltpu.VMEM((_EC,), jnp.int32),
            [pltpu.VMEM((_CS, _MCH), jnp.float32) for _ in range(2)],
            [pltpu.VMEM((_CS // 2, _NP), jnp.float32) for _ in range(2)],
            [pltpu.SemaphoreType.DMA for _ in range(5)],
        ],
    )
    def k(msg_hbm, dst_hbm, zero_hbm, out_hbm, dstv, mbufs, tables, sems):
        c = lax.axis_index("c")
        s = lax.axis_index("s")
        nch = _EC // _MCH

        @pl.when(s * _CS < wc)
        def _active():
            _scatter_body(
                msg_hbm, dst_hbm, zero_hbm, out_hbm, dstv, mbufs, tables,
                sems, c, s, nch,
            )

    def _scatter_body(msg_hbm, dst_hbm, zero_hbm, out_hbm, dstv, mbufs,
                      tables, sems, c, s, nch):

        def chunk_cp(t, buf, sem):
            return pltpu.async_copy(
                msg_hbm.at[
                    pl.ds(s * _CS, _CS), pl.ds(c * _EC + t * _MCH, _MCH)
                ],
                buf,
                sem,
            )

        zc0 = pltpu.async_copy(zero_hbm.at[pl.ds(0, 8)], tables[0], sems[2])
        zc1 = pltpu.async_copy(zero_hbm.at[pl.ds(8, 8)], tables[1], sems[3])
        dc = pltpu.async_copy(dst_hbm.at[c], dstv, sems[4])
        cps = [chunk_cp(0, mbufs[0], sems[0])]
        dc.wait()
        zc0.wait()
        zc1.wait()
        for t in range(nch):
            if t + 1 < nch:
                cps.append(chunk_cp(t + 1, mbufs[(t + 1) % 2], sems[(t + 1) % 2]))
            cps[t].wait()
            mbuf = mbufs[t % 2]

            def grp(i, _):
                d16 = dstv[pl.ds(t * _MCH + i * 16, 16)]
                for r in range(8):
                    rr = jnp.full((16,), r, jnp.int32)
                    v0 = mbuf[r, pl.ds(i * 16, 16)]
                    plsc.addupdate_scatter(tables[0], [rr, d16], v0)
                    v1 = mbuf[r + 8, pl.ds(i * 16, 16)]
                    plsc.addupdate_scatter(tables[1], [rr, d16], v1)
                return _

            lax.fori_loop(0, _MCH // 16, grp, jnp.int32(0))
        pltpu.sync_copy(
            tables[0], out_hbm.at[pl.ds(c * wc + s * _CS, _CS // 2)]
        )
        pltpu.sync_copy(
            tables[1], out_hbm.at[pl.ds(c * wc + s * _CS + 8, _CS // 2)]
        )

    return k(msgt, dst2, zrows)


def _msg_call(eap, hsrc, w1p, b1r, w2r, smat, tmat, cinp, cout, ones_cols):
    nk = 32
    wtot = cout + ones_cols

    def body(ea_ref, hs_ref, w1_ref, b1_ref, w2_ref, s_ref, t_ref, out_ref):
        eh = jnp.maximum(
            jnp.dot(ea_ref[...], w1_ref[...], preferred_element_type=jnp.float32)
            + b1_ref[...],
            0.0,
        )
        hs = hs_ref[...][:, :cinp]
        hsb = hs.astype(jnp.bfloat16)
        ehb = jnp.dot(
            eh.astype(jnp.bfloat16), s_ref[...],
            preferred_element_type=jnp.float32,
        )
        hst = jnp.dot(hsb, t_ref[...], preferred_element_type=jnp.float32)
        q = jnp.concatenate([(ehb * hst).astype(jnp.bfloat16), hsb], axis=1)
        msg = jnp.dot(q, w2_ref[...], preferred_element_type=jnp.float32)
        if ones_cols:
            msg = jnp.concatenate(
                [msg, jnp.ones((msg.shape[0], ones_cols), jnp.float32)], axis=1
            )
        out_ref[...] = msg.T

    return pl.pallas_call(
        body,
        grid=(_EP // _EB,),
        in_specs=[
            pl.BlockSpec((_EB, 8), lambda i: (i, 0)),
            pl.BlockSpec((_EB, 128), lambda i: (i, 0)),
            pl.BlockSpec((8, 32), lambda i: (0, 0)),
            pl.BlockSpec((1, 32), lambda i: (0, 0)),
            pl.BlockSpec(((nk + 1) * cinp, cout), lambda i: (0, 0)),
            pl.BlockSpec((nk, nk * cinp), lambda i: (0, 0)),
            pl.BlockSpec((cinp, nk * cinp), lambda i: (0, 0)),
        ],
        out_specs=pl.BlockSpec((wtot, _EB), lambda i: (0, i)),
        out_shape=jax.ShapeDtypeStruct((wtot, _EP), jnp.float32),
    )(eap, hsrc, w1p, b1r, w2r, smat, tmat)


def _node_call(h, rootp, parts, inv_or_cnt, biasr, gammar, betar, cinp, cout, first):
    wc = cout + (16 if first else 0)
    wout = max(cout, 128)
    nblk = _NP // _NB

    def body(h_ref, root_ref, p0_ref, p1_ref, cv_ref, bias_ref,
             g_ref, beta_ref, out_ref, inv_ref):
        p0t = p0_ref[...]
        p1t = p1_ref[...]
        psum = (p0t[:cout, :] + p1t[:cout, :]).T
        if first:
            cntt = p0t[cout : cout + 16, :] + p1t[cout : cout + 16, :]
            cnt = cntt.T
            inv = 1.0 / jnp.maximum(cnt[:, :1], 1.0)
            inv_ref[...] = jnp.broadcast_to(inv, (_NB, 16))
        else:
            inv = cv_ref[...][:, :1]
        agg = psum * inv
        y = (
            jnp.dot(h_ref[...], root_ref[...], preferred_element_type=jnp.float32)
            + agg
            + bias_ref[...]
        )
        hv = jnp.maximum(y * g_ref[...] + beta_ref[...], 0.0)
        if wout > cout:
            hv = jnp.concatenate(
                [hv, jnp.zeros((_NB, wout - cout), jnp.float32)], axis=1
            )
        out_ref[...] = hv

    in_specs = [
        pl.BlockSpec((_NB, 128), lambda i: (i, 0)),
        pl.BlockSpec((128, cout), lambda i: (0, 0)),
        pl.BlockSpec((wc, _NB), lambda i: (0, i)),
        pl.BlockSpec((wc, _NB), lambda i: (1, i)),
        pl.BlockSpec((_NB, 16), lambda i: (i, 0)),
        pl.BlockSpec((1, cout), lambda i: (0, 0)),
        pl.BlockSpec((1, cout), lambda i: (0, 0)),
        pl.BlockSpec((1, cout), lambda i: (0, 0)),
    ]
    inv_in = jnp.zeros((_NP, 16), jnp.float32) if first else inv_or_cnt
    out = pl.pallas_call(
        body,
        grid=(nblk,),
        in_specs=in_specs,
        out_specs=[
            pl.BlockSpec((_NB, wout), lambda i: (i, 0)),
            pl.BlockSpec((_NB, 16), lambda i: (i, 0)),
        ],
        out_shape=[
            jax.ShapeDtypeStruct((_NP, wout), jnp.float32),
            jax.ShapeDtypeStruct((_NP, 16), jnp.float32),
        ],
    )(h, rootp, parts, parts, inv_in, biasr, gammar, betar)
    return out


def _node_pool_call(h, rootp, parts, inv, biasr, gammar, betar, bs3, wpp, bpp):
    nblk = _NP // _NB
    cout = 256

    def body(h_ref, root_ref, p0_ref, p1_ref, cv_ref, bias_ref, g_ref,
             beta_ref, bs_ref, wp_ref, bp_ref, out_ref, acc, pcnt):
        i = pl.program_id(0)

        @pl.when(i == 0)
        def _init():
            acc[...] = jnp.zeros_like(acc)
            pcnt[...] = jnp.zeros_like(pcnt)

        psum = (p0_ref[...] + p1_ref[...]).T
        invc = cv_ref[...][:, :1]
        y = (
            jnp.dot(h_ref[...], root_ref[...], preferred_element_type=jnp.float32)
            + psum * invc
            + bias_ref[...]
        )
        h3 = jnp.maximum(y * g_ref[...] + beta_ref[...], 0.0)
        seg = lax.broadcasted_iota(jnp.int32, (_NG, _NB), 0)
        bs = bs_ref[0]
        oh = (seg == bs).astype(jnp.float32)
        acc[...] += jnp.dot(oh, h3, preferred_element_type=jnp.float32)
        pcnt[...] += jnp.broadcast_to(
            jnp.sum(oh, axis=1, keepdims=True), (_NG, 128)
        )

        @pl.when(i == nblk - 1)
        def _fin():
            pooled = acc[...] * (1.0 / jnp.maximum(pcnt[...][:, :1], 1.0))
            o = jnp.dot(
                pooled, wp_ref[...], preferred_element_type=jnp.float32
            ) + bp_ref[...]
            out_ref[...] = jnp.where(o > 0, o, 0.1 * o)

    return pl.pallas_call(
        body,
        grid=(nblk,),
        in_specs=[
            pl.BlockSpec((_NB, 128), lambda i: (i, 0)),
            pl.BlockSpec((128, cout), lambda i: (0, 0)),
            pl.BlockSpec((256, _NB), lambda i: (0, i)),
            pl.BlockSpec((256, _NB), lambda i: (1, i)),
            pl.BlockSpec((_NB, 16), lambda i: (i, 0)),
            pl.BlockSpec((1, cout), lambda i: (0, 0)),
            pl.BlockSpec((1, cout), lambda i: (0, 0)),
            pl.BlockSpec((1, cout), lambda i: (0, 0)),
            pl.BlockSpec((1, 1, _NB), lambda i: (i, 0, 0)),
            pl.BlockSpec((256, 128), lambda i: (0, 0)),
            pl.BlockSpec((1, 128), lambda i: (0, 0)),
        ],
        out_specs=pl.BlockSpec((_NG, 128), lambda i: (0, 0)),
        out_shape=jax.ShapeDtypeStruct((_NG, 128), jnp.float32),
        scratch_shapes=[
            pltpu.VMEM((_NG, 256), jnp.float32),
            pltpu.VMEM((_NG, 128), jnp.float32),
        ],
    )(h, rootp, parts, parts, inv, biasr, gammar, betar, bs3, wpp, bpp)


def _prep_layer(p, cin, cinp, cout):
    w1p = jnp.zeros((8, 32), jnp.float32).at[:3].set(p["W1"])
    b1r = p["b1"].reshape(1, 32)
    w2 = p["W2"].reshape(32, cin, cout)
    b2r = jnp.zeros((cinp, cout), jnp.float32).at[:cin].set(
        p["b2"].reshape(cin, cout)
    )
    w2r = jnp.concatenate(
        [
            jnp.zeros((32, cinp, cout), jnp.float32)
            .at[:, :cin, :]
            .set(w2)
            .reshape(32 * cinp, cout),
            b2r,
        ],
        axis=0,
    ).astype(jnp.bfloat16)
    kk = jnp.arange(32 * cinp)
    smat = (kk[None, :] // cinp == jnp.arange(32)[:, None]).astype(jnp.bfloat16)
    tmat = (kk[None, :] % cinp == jnp.arange(cinp)[:, None]).astype(jnp.bfloat16)
    rootp = jnp.zeros((128, cout), jnp.float32).at[:cin].set(p["root"])
    biasr = p["bias"].reshape(1, cout)
    gammar = (p["gamma"] / jnp.sqrt(1.0 + 1e-5)).reshape(1, cout)
    betar = p["beta"].reshape(1, cout)
    return w1p, b1r, w2r, smat, tmat, rootp, biasr, gammar, betar


def kernel(x, edge_index, edge_attr, batch_seg, params):
    f32 = jnp.float32
    src = edge_index[0]
    dst = edge_index[1]
    xp = jnp.zeros((_NP, 128), f32).at[:_N, :5].set(x)
    src3 = (
        jnp.zeros((_EP,), jnp.int32).at[:_E].set(src).reshape(_NW, _NCHUNK, _CH)
    )
    dst2 = jnp.full((_EP,), _N, jnp.int32).at[:_E].set(dst).reshape(_NC, _EC)
    eap = jnp.zeros((_EP, 8), f32).at[:_E, :3].set(edge_attr)
    bs3 = (
        jnp.full((_NP,), _NG + 8, jnp.int32)
        .at[:_N]
        .set(batch_seg)
        .reshape(_NP // _NB, 1, _NB)
    )
    zrows = jnp.zeros((_CS, _NP), f32)
    l1 = _prep_layer(params["layer1"], 5, 16, 64)
    l2 = _prep_layer(params["layer2"], 64, 64, 128)
    l3 = _prep_layer(params["layer3"], 128, 128, 256)
    wpp = jnp.zeros((256, 128), f32).at[:, :_NT].set(params["mlp_W"])
    bpp = jnp.zeros((1, 128), f32).at[0, :_NT].set(params["mlp_b"])

    w1p, b1r, w2r, smat, tmat, rootp, biasr, gammar, betar = l1
    hs = _gather_call(xp, src3)
    msg = _msg_call(eap, hs, w1p, b1r, w2r, smat, tmat, 16, 64, 16)
    parts = _scatter_call(msg, dst2, zrows, 80)
    h, inv = _node_call(xp, rootp, parts, None, biasr, gammar, betar, 16, 64, True)

    w1p, b1r, w2r, smat, tmat, rootp, biasr, gammar, betar = l2
    hs = _gather_call(h, src3)
    msg = _msg_call(eap, hs, w1p, b1r, w2r, smat, tmat, 64, 128, 0)
    parts = _scatter_call(msg, dst2, zrows, 128)
    h, _ = _node_call(h, rootp, parts, inv, biasr, gammar, betar, 64, 128, False)

    w1p, b1r, w2r, smat, tmat, rootp, biasr, gammar, betar = l3
    hs = _gather_call(h, src3)
    msg = _msg_call(eap, hs, w1p, b1r, w2r, smat, tmat, 128, 256, 0)
    parts = _scatter_call(msg, dst2, zrows, 256)

    out = _node_pool_call(
        h, rootp, parts, inv, biasr, gammar, betar, bs3, wpp, bpp
    )
    return out[:, :_NT]


# speedup vs baseline: 4.2483x; 1.1053x over previous
import functools

import jax
import jax.numpy as jnp
from jax import lax
from jax.experimental import pallas as pl
from jax.experimental.pallas import tpu as pltpu
from jax.experimental.pallas import tpu_sc as plsc

_N = 5000
_E = 10000
_NG = 256
_NT = 100

_NC = 2
_NS = 16
_NW = _NC * _NS
_CH = 80
_NCHUNK = 4
_TILE_E = _CH * _NCHUNK
_EP = _NW * _TILE_E
_NP = _NS * _TILE_E
_EB = 2048
_NB = 256


def _sc_mesh():
    return plsc.VectorSubcoreMesh(core_axis_name="c", subcore_axis_name="s")


def _gather_call(h, src3):
    cinp = 128

    @functools.partial(
        pl.kernel,
        out_type=jax.ShapeDtypeStruct((_EP, cinp), jnp.float32),
        mesh=_sc_mesh(),
        scratch_types=[
            pltpu.VMEM((_NCHUNK, _CH), jnp.int32),
            [pltpu.VMEM((_CH, cinp), jnp.float32) for _ in range(_NCHUNK)],
            [pltpu.SemaphoreType.DMA for _ in range(_NCHUNK)],
            [pltpu.SemaphoreType.DMA for _ in range(_NCHUNK)],
        ],
    )
    def k(h_hbm, src_hbm, out_hbm, idx_v, rows, gsems, wsems):
        c = lax.axis_index("c")
        s = lax.axis_index("s")
        wid = s * _NC + c
        pltpu.sync_copy(src_hbm.at[wid], idx_v)
        gcps = [
            pltpu.async_copy(h_hbm.at[idx_v.at[j]], rows[j], gsems[j])
            for j in range(_NCHUNK)
        ]
        wcps = []
        for j in range(_NCHUNK):
            gcps[j].wait()
            wcps.append(
                pltpu.async_copy(
                    rows[j],
                    out_hbm.at[pl.ds(wid * _TILE_E + j * _CH, _CH)],
                    wsems[j],
                )
            )
        for w in wcps:
            w.wait()

    return k(h, src3)


_EC = _EP // _NC
_CS = 16
_MCH = 1024


def _scatter_call(msgt, dst2, zrows, wc):
    @functools.partial(
        pl.kernel,
        out_type=jax.ShapeDtypeStruct((2 * wc, _NP), jnp.float32),
        mesh=_sc_mesh(),
        compiler_params=pltpu.CompilerParams(needs_layout_passes=False),
        scratch_types=[
            pltpu.VMEM((_EC,), jnp.int32),
            [pltpu.VMEM((_CS, _MCH), jnp.float32) for _ in range(2)],
            [pltpu.VMEM((_CS // 2, _NP), jnp.float32) for _ in range(2)],
            [pltpu.SemaphoreType.DMA for _ in range(5)],
        ],
    )
    def k(msg_hbm, dst_hbm, zero_hbm, out_hbm, dstv, mbufs, tables, sems):
        c = lax.axis_index("c")
        s = lax.axis_index("s")
        nch = _EC // _MCH

        @pl.when(s * _CS < wc)
        def _active():
            _scatter_body(
                msg_hbm, dst_hbm, zero_hbm, out_hbm, dstv, mbufs, tables,
                sems, c, s, nch,
            )

    def _scatter_body(msg_hbm, dst_hbm, zero_hbm, out_hbm, dstv, mbufs,
                      tables, sems, c, s, nch):

        def chunk_cp(t, buf, sem):
            return pltpu.async_copy(
                msg_hbm.at[
                    pl.ds(s * _CS, _CS), pl.ds(c * _EC + t * _MCH, _MCH)
                ],
                buf,
                sem,
            )

        zc0 = pltpu.async_copy(zero_hbm.at[pl.ds(0, 8)], tables[0], sems[2])
        zc1 = pltpu.async_copy(zero_hbm.at[pl.ds(8, 8)], tables[1], sems[3])
        dc = pltpu.async_copy(dst_hbm.at[c], dstv, sems[4])
        cps = [chunk_cp(0, mbufs[0], sems[0])]
        dc.wait()
        zc0.wait()
        zc1.wait()
        for t in range(nch):
            if t + 1 < nch:
                cps.append(chunk_cp(t + 1, mbufs[(t + 1) % 2], sems[(t + 1) % 2]))
            cps[t].wait()
            mbuf = mbufs[t % 2]

            def grp(i, _):
                d16 = dstv[pl.ds(t * _MCH + i * 16, 16)]
                for r in range(8):
                    rr = jnp.full((16,), r, jnp.int32)
                    v0 = mbuf[r, pl.ds(i * 16, 16)]
                    plsc.addupdate_scatter(tables[0], [rr, d16], v0)
                    v1 = mbuf[r + 8, pl.ds(i * 16, 16)]
                    plsc.addupdate_scatter(tables[1], [rr, d16], v1)
                return _

            lax.fori_loop(0, _MCH // 16, grp, jnp.int32(0))
        pltpu.sync_copy(
            tables[0], out_hbm.at[pl.ds(c * wc + s * _CS, _CS // 2)]
        )
        pltpu.sync_copy(
            tables[1], out_hbm.at[pl.ds(c * wc + s * _CS + 8, _CS // 2)]
        )

    return k(msgt, dst2, zrows)


def _msg_call(eap, hsrc, w1p, b1r, w2r, smat, tmat, cinp, cout, ones_cols):
    nk = 32
    wtot = cout + ones_cols

    def body(ea_ref, hs_ref, w1_ref, b1_ref, w2_ref, s_ref, t_ref, out_ref):
        eh = jnp.maximum(
            jnp.dot(ea_ref[...], w1_ref[...], preferred_element_type=jnp.float32)
            + b1_ref[...],
            0.0,
        )
        hs = hs_ref[...][:, :cinp]
        hsb = hs.astype(jnp.bfloat16)
        if cinp == 128:
            q3 = eh.astype(jnp.bfloat16)[:, :, None] * hsb[:, None, :]
            qm = q3.reshape(_EB, nk * cinp)
        else:
            ehb = jnp.dot(
                eh.astype(jnp.bfloat16), s_ref[...],
                preferred_element_type=jnp.float32,
            )
            hst = jnp.dot(hsb, t_ref[...], preferred_element_type=jnp.float32)
            qm = (ehb * hst).astype(jnp.bfloat16)
        q = jnp.concatenate([qm, hsb], axis=1)
        msg = jnp.dot(q, w2_ref[...], preferred_element_type=jnp.float32)
        if ones_cols:
            msg = jnp.concatenate(
                [msg, jnp.ones((msg.shape[0], ones_cols), jnp.float32)], axis=1
            )
        out_ref[...] = msg.T

    return pl.pallas_call(
        body,
        grid=(_EP // _EB,),
        in_specs=[
            pl.BlockSpec((_EB, 8), lambda i: (i, 0)),
            pl.BlockSpec((_EB, 128), lambda i: (i, 0)),
            pl.BlockSpec((8, 32), lambda i: (0, 0)),
            pl.BlockSpec((1, 32), lambda i: (0, 0)),
            pl.BlockSpec(((nk + 1) * cinp, cout), lambda i: (0, 0)),
            pl.BlockSpec((nk, nk * cinp), lambda i: (0, 0)),
            pl.BlockSpec((cinp, nk * cinp), lambda i: (0, 0)),
        ],
        out_specs=pl.BlockSpec((wtot, _EB), lambda i: (0, i)),
        out_shape=jax.ShapeDtypeStruct((wtot, _EP), jnp.float32),
    )(eap, hsrc, w1p, b1r, w2r, smat, tmat)


def _node_call(h, rootp, parts, inv_or_cnt, biasr, gammar, betar, cinp, cout, first):
    wc = cout + (16 if first else 0)
    wout = max(cout, 128)
    nblk = _NP // _NB

    def body(h_ref, root_ref, p0_ref, p1_ref, cv_ref, bias_ref,
             g_ref, beta_ref, out_ref, inv_ref):
        p0t = p0_ref[...]
        p1t = p1_ref[...]
        psum = (p0t[:cout, :] + p1t[:cout, :]).T
        if first:
            cntt = p0t[cout : cout + 16, :] + p1t[cout : cout + 16, :]
            cnt = cntt.T
            inv = 1.0 / jnp.maximum(cnt[:, :1], 1.0)
            inv_ref[...] = jnp.broadcast_to(inv, (_NB, 16))
        else:
            inv = cv_ref[...][:, :1]
        agg = psum * inv
        y = (
            jnp.dot(h_ref[...], root_ref[...], preferred_element_type=jnp.float32)
            + agg
            + bias_ref[...]
        )
        hv = jnp.maximum(y * g_ref[...] + beta_ref[...], 0.0)
        if wout > cout:
            hv = jnp.concatenate(
                [hv, jnp.zeros((_NB, wout - cout), jnp.float32)], axis=1
            )
        out_ref[...] = hv

    in_specs = [
        pl.BlockSpec((_NB, 128), lambda i: (i, 0)),
        pl.BlockSpec((128, cout), lambda i: (0, 0)),
        pl.BlockSpec((wc, _NB), lambda i: (0, i)),
        pl.BlockSpec((wc, _NB), lambda i: (1, i)),
        pl.BlockSpec((_NB, 16), lambda i: (i, 0)),
        pl.BlockSpec((1, cout), lambda i: (0, 0)),
        pl.BlockSpec((1, cout), lambda i: (0, 0)),
        pl.BlockSpec((1, cout), lambda i: (0, 0)),
    ]
    inv_in = jnp.zeros((_NP, 16), jnp.float32) if first else inv_or_cnt
    out = pl.pallas_call(
        body,
        grid=(nblk,),
        in_specs=in_specs,
        out_specs=[
            pl.BlockSpec((_NB, wout), lambda i: (i, 0)),
            pl.BlockSpec((_NB, 16), lambda i: (i, 0)),
        ],
        out_shape=[
            jax.ShapeDtypeStruct((_NP, wout), jnp.float32),
            jax.ShapeDtypeStruct((_NP, 16), jnp.float32),
        ],
    )(h, rootp, parts, parts, inv_in, biasr, gammar, betar)
    return out


def _node_pool_call(h, rootp, parts, inv, biasr, gammar, betar, bs3, wpp, bpp):
    nblk = _NP // _NB
    cout = 256

    def body(h_ref, root_ref, p0_ref, p1_ref, cv_ref, bias_ref, g_ref,
             beta_ref, bs_ref, wp_ref, bp_ref, out_ref, acc, pcnt):
        i = pl.program_id(0)

        @pl.when(i == 0)
        def _init():
            acc[...] = jnp.zeros_like(acc)
            pcnt[...] = jnp.zeros_like(pcnt)

        psum = (p0_ref[...] + p1_ref[...]).T
        invc = cv_ref[...][:, :1]
        y = (
            jnp.dot(h_ref[...], root_ref[...], preferred_element_type=jnp.float32)
            + psum * invc
            + bias_ref[...]
        )
        h3 = jnp.maximum(y * g_ref[...] + beta_ref[...], 0.0)
        seg = lax.broadcasted_iota(jnp.int32, (_NG, _NB), 0)
        bs = bs_ref[0]
        oh = (seg == bs).astype(jnp.float32)
        acc[...] += jnp.dot(oh, h3, preferred_element_type=jnp.float32)
        pcnt[...] += jnp.broadcast_to(
            jnp.sum(oh, axis=1, keepdims=True), (_NG, 128)
        )

        @pl.when(i == nblk - 1)
        def _fin():
            pooled = acc[...] * (1.0 / jnp.maximum(pcnt[...][:, :1], 1.0))
            o = jnp.dot(
                pooled, wp_ref[...], preferred_element_type=jnp.float32
            ) + bp_ref[...]
            out_ref[...] = jnp.where(o > 0, o, 0.1 * o)

    return pl.pallas_call(
        body,
        grid=(nblk,),
        in_specs=[
            pl.BlockSpec((_NB, 128), lambda i: (i, 0)),
            pl.BlockSpec((128, cout), lambda i: (0, 0)),
            pl.BlockSpec((256, _NB), lambda i: (0, i)),
            pl.BlockSpec((256, _NB), lambda i: (1, i)),
            pl.BlockSpec((_NB, 16), lambda i: (i, 0)),
            pl.BlockSpec((1, cout), lambda i: (0, 0)),
            pl.BlockSpec((1, cout), lambda i: (0, 0)),
            pl.BlockSpec((1, cout), lambda i: (0, 0)),
            pl.BlockSpec((1, 1, _NB), lambda i: (i, 0, 0)),
            pl.BlockSpec((256, 128), lambda i: (0, 0)),
            pl.BlockSpec((1, 128), lambda i: (0, 0)),
        ],
        out_specs=pl.BlockSpec((_NG, 128), lambda i: (0, 0)),
        out_shape=jax.ShapeDtypeStruct((_NG, 128), jnp.float32),
        scratch_shapes=[
            pltpu.VMEM((_NG, 256), jnp.float32),
            pltpu.VMEM((_NG, 128), jnp.float32),
        ],
    )(h, rootp, parts, parts, inv, biasr, gammar, betar, bs3, wpp, bpp)


def _prep_layer(p, cin, cinp, cout):
    w1p = jnp.zeros((8, 32), jnp.float32).at[:3].set(p["W1"])
    b1r = p["b1"].reshape(1, 32)
    w2 = p["W2"].reshape(32, cin, cout)
    b2r = jnp.zeros((cinp, cout), jnp.float32).at[:cin].set(
        p["b2"].reshape(cin, cout)
    )
    w2r = jnp.concatenate(
        [
            jnp.zeros((32, cinp, cout), jnp.float32)
            .at[:, :cin, :]
            .set(w2)
            .reshape(32 * cinp, cout),
            b2r,
        ],
        axis=0,
    ).astype(jnp.bfloat16)
    kk = jnp.arange(32 * cinp)
    smat = (kk[None, :] // cinp == jnp.arange(32)[:, None]).astype(jnp.bfloat16)
    tmat = (kk[None, :] % cinp == jnp.arange(cinp)[:, None]).astype(jnp.bfloat16)
    rootp = jnp.zeros((128, cout), jnp.float32).at[:cin].set(p["root"])
    biasr = p["bias"].reshape(1, cout)
    gammar = (p["gamma"] / jnp.sqrt(1.0 + 1e-5)).reshape(1, cout)
    betar = p["beta"].reshape(1, cout)
    return w1p, b1r, w2r, smat, tmat, rootp, biasr, gammar, betar


def kernel(x, edge_index, edge_attr, batch_seg, params):
    f32 = jnp.float32
    src = edge_index[0]
    dst = edge_index[1]
    xp = jnp.zeros((_NP, 128), f32).at[:_N, :5].set(x)
    src3 = (
        jnp.zeros((_EP,), jnp.int32).at[:_E].set(src).reshape(_NW, _NCHUNK, _CH)
    )
    dst2 = jnp.full((_EP,), _N, jnp.int32).at[:_E].set(dst).reshape(_NC, _EC)
    eap = jnp.zeros((_EP, 8), f32).at[:_E, :3].set(edge_attr)
    bs3 = (
        jnp.full((_NP,), _NG + 8, jnp.int32)
        .at[:_N]
        .set(batch_seg)
        .reshape(_NP // _NB, 1, _NB)
    )
    zrows = jnp.zeros((_CS, _NP), f32)
    l1 = _prep_layer(params["layer1"], 5, 16, 64)
    l2 = _prep_layer(params["layer2"], 64, 64, 128)
    l3 = _prep_layer(params["layer3"], 128, 128, 256)
    wpp = jnp.zeros((256, 128), f32).at[:, :_NT].set(params["mlp_W"])
    bpp = jnp.zeros((1, 128), f32).at[0, :_NT].set(params["mlp_b"])

    w1p, b1r, w2r, smat, tmat, rootp, biasr, gammar, betar = l1
    hs = _gather_call(xp, src3)
    msg = _msg_call(eap, hs, w1p, b1r, w2r, smat, tmat, 16, 64, 16)
    parts = _scatter_call(msg, dst2, zrows, 80)
    h, inv = _node_call(xp, rootp, parts, None, biasr, gammar, betar, 16, 64, True)

    w1p, b1r, w2r, smat, tmat, rootp, biasr, gammar, betar = l2
    hs = _gather_call(h, src3)
    msg = _msg_call(eap, hs, w1p, b1r, w2r, smat, tmat, 64, 128, 0)
    parts = _scatter_call(msg, dst2, zrows, 128)
    h, _ = _node_call(h, rootp, parts, inv, biasr, gammar, betar, 64, 128, False)

    w1p, b1r, w2r, smat, tmat, rootp, biasr, gammar, betar = l3
    hs = _gather_call(h, src3)
    msg = _msg_call(eap, hs, w1p, b1r, w2r, smat, tmat, 128, 256, 0)
    parts = _scatter_call(msg, dst2, zrows, 256)

    out = _node_pool_call(
        h, rootp, parts, inv, biasr, gammar, betar, bs3, wpp, bpp
    )
    return out[:, :_NT]


# speedup vs baseline: 4.4409x; 1.0453x over previous
import functools

import jax
import jax.numpy as jnp
from jax import lax
from jax.experimental import pallas as pl
from jax.experimental.pallas import tpu as pltpu
from jax.experimental.pallas import tpu_sc as plsc

_N = 5000
_E = 10000
_NG = 256
_NT = 100

_NC = 2
_NS = 16
_NW = _NC * _NS
_CH = 80
_NCHUNK = 4
_TILE_E = _CH * _NCHUNK
_EP = _NW * _TILE_E
_NP = _NS * _TILE_E
_EB = 2048
_NB = 256


def _sc_mesh():
    return plsc.VectorSubcoreMesh(core_axis_name="c", subcore_axis_name="s")


def _gather_call(h, src3):
    cinp = 128

    @functools.partial(
        pl.kernel,
        out_type=jax.ShapeDtypeStruct((_EP, cinp), jnp.float32),
        mesh=_sc_mesh(),
        scratch_types=[
            pltpu.VMEM((_NCHUNK, _CH), jnp.int32),
            [pltpu.VMEM((_CH, cinp), jnp.float32) for _ in range(_NCHUNK)],
            [pltpu.SemaphoreType.DMA for _ in range(_NCHUNK)],
            [pltpu.SemaphoreType.DMA for _ in range(_NCHUNK)],
        ],
    )
    def k(h_hbm, src_hbm, out_hbm, idx_v, rows, gsems, wsems):
        c = lax.axis_index("c")
        s = lax.axis_index("s")
        wid = s * _NC + c
        pltpu.sync_copy(src_hbm.at[wid], idx_v)
        gcps = [
            pltpu.async_copy(h_hbm.at[idx_v.at[j]], rows[j], gsems[j])
            for j in range(_NCHUNK)
        ]
        wcps = []
        for j in range(_NCHUNK):
            gcps[j].wait()
            wcps.append(
                pltpu.async_copy(
                    rows[j],
                    out_hbm.at[pl.ds(wid * _TILE_E + j * _CH, _CH)],
                    wsems[j],
                )
            )
        for w in wcps:
            w.wait()

    return k(h, src3)


_EC = _EP // _NC
_CS = 16
_MCH = 1024


def _scatter_call(msgt, dstp, zrows, wc, ngroups, ech, mch):
    nstripes = wc // _CS
    nch = ech // mch

    @functools.partial(
        pl.kernel,
        out_type=jax.ShapeDtypeStruct((ngroups * wc, _NP), jnp.float32),
        mesh=_sc_mesh(),
        compiler_params=pltpu.CompilerParams(needs_layout_passes=False),
        scratch_types=[
            pltpu.VMEM((ech,), jnp.int32),
            [pltpu.VMEM((_CS, mch), jnp.float32) for _ in range(2)],
            [pltpu.VMEM((_CS // 2, _NP), jnp.float32) for _ in range(2)],
            [pltpu.SemaphoreType.DMA for _ in range(5)],
        ],
    )
    def k(msg_hbm, dst_hbm, zero_hbm, out_hbm, dstv, mbufs, tables, sems):
        c = lax.axis_index("c")
        s = lax.axis_index("s")
        wid = s * _NC + c
        g = wid // nstripes
        st = wid % nstripes

        @pl.when(g < ngroups)
        def _active():
            _scatter_body(
                msg_hbm, dst_hbm, zero_hbm, out_hbm, dstv, mbufs, tables,
                sems, g, st,
            )

    def _scatter_body(msg_hbm, dst_hbm, zero_hbm, out_hbm, dstv, mbufs,
                      tables, sems, g, st):

        def chunk_cp(t, buf, sem):
            return pltpu.async_copy(
                msg_hbm.at[
                    pl.ds(st * _CS, _CS), pl.ds(g * ech + t * mch, mch)
                ],
                buf,
                sem,
            )

        zc0 = pltpu.async_copy(zero_hbm.at[pl.ds(0, 8)], tables[0], sems[2])
        zc1 = pltpu.async_copy(zero_hbm.at[pl.ds(8, 8)], tables[1], sems[3])
        dc = pltpu.async_copy(dst_hbm.at[pl.ds(g * ech, ech)], dstv, sems[4])
        cps = [chunk_cp(0, mbufs[0], sems[0])]
        dc.wait()
        zc0.wait()
        zc1.wait()
        for t in range(nch):
            if t + 1 < nch:
                cps.append(chunk_cp(t + 1, mbufs[(t + 1) % 2], sems[(t + 1) % 2]))
            cps[t].wait()
            mbuf = mbufs[t % 2]

            def grp(i, _):
                d16 = dstv[pl.ds(t * mch + i * 16, 16)]
                for r in range(8):
                    rr = jnp.full((16,), r, jnp.int32)
                    v0 = mbuf[r, pl.ds(i * 16, 16)]
                    plsc.addupdate_scatter(tables[0], [rr, d16], v0)
                    v1 = mbuf[r + 8, pl.ds(i * 16, 16)]
                    plsc.addupdate_scatter(tables[1], [rr, d16], v1)
                return _

            lax.fori_loop(0, mch // 16, grp, jnp.int32(0))
        pltpu.sync_copy(
            tables[0], out_hbm.at[pl.ds(g * wc + st * _CS, _CS // 2)]
        )
        pltpu.sync_copy(
            tables[1], out_hbm.at[pl.ds(g * wc + st * _CS + 8, _CS // 2)]
        )

    return k(msgt, dstp, zrows)


def _msg_call(eap, hsrc, w1p, b1r, w2r, smat, tmat, cinp, cout, ones_cols):
    nk = 32
    wtot = cout + ones_cols

    def body(ea_ref, hs_ref, w1_ref, b1_ref, w2_ref, s_ref, t_ref, out_ref):
        eh = jnp.maximum(
            jnp.dot(ea_ref[...], w1_ref[...], preferred_element_type=jnp.float32)
            + b1_ref[...],
            0.0,
        )
        hs = hs_ref[...][:, :cinp]
        hsb = hs.astype(jnp.bfloat16)
        if cinp == 128:
            q3 = eh.astype(jnp.bfloat16)[:, :, None] * hsb[:, None, :]
            qm = q3.reshape(_EB, nk * cinp)
        else:
            ehb = jnp.dot(
                eh.astype(jnp.bfloat16), s_ref[...],
                preferred_element_type=jnp.float32,
            )
            hst = jnp.dot(hsb, t_ref[...], preferred_element_type=jnp.float32)
            qm = (ehb * hst).astype(jnp.bfloat16)
        q = jnp.concatenate([qm, hsb], axis=1)
        msg = jnp.dot(q, w2_ref[...], preferred_element_type=jnp.float32)
        if ones_cols:
            msg = jnp.concatenate(
                [msg, jnp.ones((msg.shape[0], ones_cols), jnp.float32)], axis=1
            )
        out_ref[...] = msg.T

    return pl.pallas_call(
        body,
        grid=(_EP // _EB,),
        in_specs=[
            pl.BlockSpec((_EB, 8), lambda i: (i, 0)),
            pl.BlockSpec((_EB, 128), lambda i: (i, 0)),
            pl.BlockSpec((8, 32), lambda i: (0, 0)),
            pl.BlockSpec((1, 32), lambda i: (0, 0)),
            pl.BlockSpec(((nk + 1) * cinp, cout), lambda i: (0, 0)),
            pl.BlockSpec((nk, nk * cinp), lambda i: (0, 0)),
            pl.BlockSpec((cinp, nk * cinp), lambda i: (0, 0)),
        ],
        out_specs=pl.BlockSpec((wtot, _EB), lambda i: (0, i)),
        out_shape=jax.ShapeDtypeStruct((wtot, _EP), jnp.float32),
    )(eap, hsrc, w1p, b1r, w2r, smat, tmat)


def _node_call(h, rootp, parts, inv_or_cnt, biasr, gammar, betar, cinp, cout,
               first, ngroups):
    wc = cout + (16 if first else 0)
    wout = max(cout, 128)
    nblk = _NP // _NB

    def body(h_ref, root_ref, *refs):
        p_refs = refs[:ngroups]
        cv_ref, bias_ref, g_ref, beta_ref, out_ref, inv_ref = refs[ngroups:]
        pts = [p[...] for p in p_refs]
        psumt = pts[0][:cout, :]
        for p in pts[1:]:
            psumt = psumt + p[:cout, :]
        psum = psumt.T
        if first:
            cntt = pts[0][cout : cout + 16, :]
            for p in pts[1:]:
                cntt = cntt + p[cout : cout + 16, :]
            cnt = cntt.T
            inv = 1.0 / jnp.maximum(cnt[:, :1], 1.0)
            inv_ref[...] = jnp.broadcast_to(inv, (_NB, 16))
        else:
            inv = cv_ref[...][:, :1]
        agg = psum * inv
        y = (
            jnp.dot(h_ref[...], root_ref[...], preferred_element_type=jnp.float32)
            + agg
            + bias_ref[...]
        )
        hv = jnp.maximum(y * g_ref[...] + beta_ref[...], 0.0)
        if wout > cout:
            hv = jnp.concatenate(
                [hv, jnp.zeros((_NB, wout - cout), jnp.float32)], axis=1
            )
        out_ref[...] = hv

    in_specs = [
        pl.BlockSpec((_NB, 128), lambda i: (i, 0)),
        pl.BlockSpec((128, cout), lambda i: (0, 0)),
    ] + [
        pl.BlockSpec((wc, _NB), lambda i, g=g: (g, i)) for g in range(ngroups)
    ] + [
        pl.BlockSpec((_NB, 16), lambda i: (i, 0)),
        pl.BlockSpec((1, cout), lambda i: (0, 0)),
        pl.BlockSpec((1, cout), lambda i: (0, 0)),
        pl.BlockSpec((1, cout), lambda i: (0, 0)),
    ]
    inv_in = jnp.zeros((_NP, 16), jnp.float32) if first else inv_or_cnt
    out = pl.pallas_call(
        body,
        grid=(nblk,),
        in_specs=in_specs,
        out_specs=[
            pl.BlockSpec((_NB, wout), lambda i: (i, 0)),
            pl.BlockSpec((_NB, 16), lambda i: (i, 0)),
        ],
        out_shape=[
            jax.ShapeDtypeStruct((_NP, wout), jnp.float32),
            jax.ShapeDtypeStruct((_NP, 16), jnp.float32),
        ],
    )(h, rootp, *([parts] * ngroups), inv_in, biasr, gammar, betar)
    return out


def _node_pool_call(h, rootp, parts, inv, biasr, gammar, betar, bs3, wpp, bpp):
    nblk = _NP // _NB
    cout = 256

    def body(h_ref, root_ref, p0_ref, p1_ref, cv_ref, bias_ref, g_ref,
             beta_ref, bs_ref, wp_ref, bp_ref, out_ref, acc, pcnt):
        i = pl.program_id(0)

        @pl.when(i == 0)
        def _init():
            acc[...] = jnp.zeros_like(acc)
            pcnt[...] = jnp.zeros_like(pcnt)

        psum = (p0_ref[...] + p1_ref[...]).T
        invc = cv_ref[...][:, :1]
        y = (
            jnp.dot(h_ref[...], root_ref[...], preferred_element_type=jnp.float32)
            + psum * invc
            + bias_ref[...]
        )
        h3 = jnp.maximum(y * g_ref[...] + beta_ref[...], 0.0)
        seg = lax.broadcasted_iota(jnp.int32, (_NG, _NB), 0)
        bs = bs_ref[0]
        oh = (seg == bs).astype(jnp.float32)
        acc[...] += jnp.dot(oh, h3, preferred_element_type=jnp.float32)
        pcnt[...] += jnp.broadcast_to(
            jnp.sum(oh, axis=1, keepdims=True), (_NG, 128)
        )

        @pl.when(i == nblk - 1)
        def _fin():
            pooled = acc[...] * (1.0 / jnp.maximum(pcnt[...][:, :1], 1.0))
            o = jnp.dot(
                pooled, wp_ref[...], preferred_element_type=jnp.float32
            ) + bp_ref[...]
            out_ref[...] = jnp.where(o > 0, o, 0.1 * o)

    return pl.pallas_call(
        body,
        grid=(nblk,),
        in_specs=[
            pl.BlockSpec((_NB, 128), lambda i: (i, 0)),
            pl.BlockSpec((128, cout), lambda i: (0, 0)),
            pl.BlockSpec((256, _NB), lambda i: (0, i)),
            pl.BlockSpec((256, _NB), lambda i: (1, i)),
            pl.BlockSpec((_NB, 16), lambda i: (i, 0)),
            pl.BlockSpec((1, cout), lambda i: (0, 0)),
            pl.BlockSpec((1, cout), lambda i: (0, 0)),
            pl.BlockSpec((1, cout), lambda i: (0, 0)),
            pl.BlockSpec((1, 1, _NB), lambda i: (i, 0, 0)),
            pl.BlockSpec((256, 128), lambda i: (0, 0)),
            pl.BlockSpec((1, 128), lambda i: (0, 0)),
        ],
        out_specs=pl.BlockSpec((_NG, 128), lambda i: (0, 0)),
        out_shape=jax.ShapeDtypeStruct((_NG, 128), jnp.float32),
        scratch_shapes=[
            pltpu.VMEM((_NG, 256), jnp.float32),
            pltpu.VMEM((_NG, 128), jnp.float32),
        ],
    )(h, rootp, parts, parts, inv, biasr, gammar, betar, bs3, wpp, bpp)


def _prep_layer(p, cin, cinp, cout):
    w1p = jnp.zeros((8, 32), jnp.float32).at[:3].set(p["W1"])
    b1r = p["b1"].reshape(1, 32)
    w2 = p["W2"].reshape(32, cin, cout)
    b2r = jnp.zeros((cinp, cout), jnp.float32).at[:cin].set(
        p["b2"].reshape(cin, cout)
    )
    w2r = jnp.concatenate(
        [
            jnp.zeros((32, cinp, cout), jnp.float32)
            .at[:, :cin, :]
            .set(w2)
            .reshape(32 * cinp, cout),
            b2r,
        ],
        axis=0,
    ).astype(jnp.bfloat16)
    kk = jnp.arange(32 * cinp)
    smat = (kk[None, :] // cinp == jnp.arange(32)[:, None]).astype(jnp.bfloat16)
    tmat = (kk[None, :] % cinp == jnp.arange(cinp)[:, None]).astype(jnp.bfloat16)
    rootp = jnp.zeros((128, cout), jnp.float32).at[:cin].set(p["root"])
    biasr = p["bias"].reshape(1, cout)
    gammar = (p["gamma"] / jnp.sqrt(1.0 + 1e-5)).reshape(1, cout)
    betar = p["beta"].reshape(1, cout)
    return w1p, b1r, w2r, smat, tmat, rootp, biasr, gammar, betar


def kernel(x, edge_index, edge_attr, batch_seg, params):
    f32 = jnp.float32
    src = edge_index[0]
    dst = edge_index[1]
    xp = jnp.zeros((_NP, 128), f32).at[:_N, :5].set(x)
    src3 = (
        jnp.zeros((_EP,), jnp.int32).at[:_E].set(src).reshape(_NW, _NCHUNK, _CH)
    )
    dstp = jnp.full((_EP,), _N, jnp.int32).at[:_E].set(dst)
    eap = jnp.zeros((_EP, 8), f32).at[:_E, :3].set(edge_attr)
    bs3 = (
        jnp.full((_NP,), _NG + 8, jnp.int32)
        .at[:_N]
        .set(batch_seg)
        .reshape(_NP // _NB, 1, _NB)
    )
    zrows = jnp.zeros((_CS, _NP), f32)
    l1 = _prep_layer(params["layer1"], 5, 16, 64)
    l2 = _prep_layer(params["layer2"], 64, 64, 128)
    l3 = _prep_layer(params["layer3"], 128, 128, 256)
    wpp = jnp.zeros((256, 128), f32).at[:, :_NT].set(params["mlp_W"])
    bpp = jnp.zeros((1, 128), f32).at[0, :_NT].set(params["mlp_b"])

    w1p, b1r, w2r, smat, tmat, rootp, biasr, gammar, betar = l1
    hs = _gather_call(xp, src3)
    msg = _msg_call(eap, hs, w1p, b1r, w2r, smat, tmat, 16, 64, 16)
    parts = _scatter_call(msg, dstp, zrows, 80, 5, 2048, 1024)
    h, inv = _node_call(
        xp, rootp, parts, None, biasr, gammar, betar, 16, 64, True, 5
    )

    w1p, b1r, w2r, smat, tmat, rootp, biasr, gammar, betar = l2
    hs = _gather_call(h, src3)
    msg = _msg_call(eap, hs, w1p, b1r, w2r, smat, tmat, 64, 128, 0)
    parts = _scatter_call(msg, dstp, zrows, 128, 4, 2560, 512)
    h, _ = _node_call(
        h, rootp, parts, inv, biasr, gammar, betar, 64, 128, False, 4
    )

    w1p, b1r, w2r, smat, tmat, rootp, biasr, gammar, betar = l3
    hs = _gather_call(h, src3)
    msg = _msg_call(eap, hs, w1p, b1r, w2r, smat, tmat, 128, 256, 0)
    parts = _scatter_call(msg, dstp, zrows, 256, 2, 5120, 1024)

    out = _node_pool_call(
        h, rootp, parts, inv, biasr, gammar, betar, bs3, wpp, bpp
    )
    return out[:, :_NT]
